# baseline (speedup 1.0000x reference)
import functools
import jax
import jax.numpy as jnp
from jax import lax
from jax.experimental import pallas as pl
from jax.experimental.pallas import tpu as pltpu
from jax.experimental.pallas import tpu_sc as plsc

N_NODES = 10000
N_EDGES = 160000
LAT = 128
HID = 16
CORE_STEPS = 3

BN = 1000
BE = 2000
NW = 32
EPW = N_EDGES // NW
NPT = N_NODES // 16
CH = 200
NCH = EPW // CH

_EPS = 1e-5


def _ln(h, g, bt):
    mu = jnp.mean(h, axis=-1, keepdims=True)
    var = jnp.mean((h - mu) * (h - mu), axis=-1, keepdims=True)
    return (h - mu) * lax.rsqrt(var + _EPS) * g + bt


def _dot(a, b):
    return jnp.dot(a, b, preferred_element_type=jnp.float32)


def _mlp_tail(h1pre, W2, b2, g, bt):
    h = jnp.maximum(h1pre, 0.0)
    h = jnp.maximum(_dot(h, W2) + b2, 0.0)
    return _ln(h, g, bt)


def _row_spec(b, d):
    return pl.BlockSpec((b, d), lambda i: (i, 0))


def _w_spec(shape):
    return pl.BlockSpec(shape, lambda i: tuple(0 for _ in shape))


def _enc_node_body(x, W1, b1, W2, b2, g, bt, Avs, Avd, Bv, Av0s, Av0d, Bv0,
                   ns_o, nd_o, pn_o, fs_o, fd_o, fn_o):
    h1 = _dot(x[...], W1[...]) + b1[...]
    v0 = _mlp_tail(h1, W2[...], b2[...], g[...], bt[...])
    fs = _dot(v0, Av0s[...])
    fd = _dot(v0, Av0d[...])
    fn = _dot(v0, Bv0[...])
    fs_o[...] = fs
    fd_o[...] = fd
    fn_o[...] = fn
    ns_o[...] = fs + _dot(v0, Avs[...])
    nd_o[...] = fd + _dot(v0, Avd[...])
    pn_o[...] = fn + _dot(v0, Bv[...])


def _enc_edge_body(x, W1, b1, W2, b2, g, bt, A0, A1, pe0_o, pec_o):
    h1 = _dot(x[...], W1[...]) + b1[...]
    e0 = _mlp_tail(h1, W2[...], b2[...], g[...], bt[...])
    pe0_o[...] = _dot(e0, A0[...])
    pec_o[...] = _dot(e0, A1[...])


def _edge_step_body(pe0, pec, gs, gd, b1, W2, b2, g, bt, Ae,
                    en_o, pec_o):
    h1 = pe0[...] + pec[...] + gs[...] + gd[...] + b1[...]
    en = _mlp_tail(h1, W2[...], b2[...], g[...], bt[...])
    en_o[...] = en
    pec_o[...] = _dot(en, Ae[...])


def _edge_core_body(pe0, pec, gs, gd, b1, W2, b2, g, bt, en_o):
    h1 = pe0[...] + pec[...] + gs[...] + gd[...] + b1[...]
    en_o[...] = _mlp_tail(h1, W2[...], b2[...], g[...], bt[...])


def _edge_dec_body(en, dW1, db1, dW2, db2, dg, dbt, oW, ob, eout_o):
    d1 = _dot(en[...], dW1[...]) + db1[...]
    dec = _mlp_tail(d1, dW2[...], db2[...], dg[...], dbt[...])
    eout_o[...] = _dot(dec, oW[...]) + ob[...]


def _node_step_body(pn, a0, a1, a2, a3, c0, c1, fs, fd, fn, b1, W2, b2, g, bt,
                    Bagg, Avs, Avd, Bv, ns_o, nd_o, pn_o):
    cm = jnp.maximum(c0[...] + c1[...], 1.0)[:, 0:1]
    agg = (a0[...] + a1[...] + a2[...] + a3[...]) / cm
    h1 = pn[...] + _dot(agg, Bagg[...]) + b1[...]
    vn = _mlp_tail(h1, W2[...], b2[...], g[...], bt[...])
    ns_o[...] = fs[...] + _dot(vn, Avs[...])
    nd_o[...] = fd[...] + _dot(vn, Avd[...])
    pn_o[...] = fn[...] + _dot(vn, Bv[...])


def _node_last_body(pn, a0, a1, c0, c1, b1, W2, b2, g, bt, Bagg,
                    dW1, db1, dW2, db2, dg, dbt, oW, ob, vout_o):
    cm = jnp.maximum(c0[...] + c1[...], 1.0)[:, 0:1]
    agg = (a0[...] + a1[...]) / cm
    h1 = pn[...] + _dot(agg, Bagg[...]) + b1[...]
    vn = _mlp_tail(h1, W2[...], b2[...], g[...], bt[...])
    d1 = _dot(vn, dW1[...]) + db1[...]
    dec = _mlp_tail(d1, dW2[...], db2[...], dg[...], dbt[...])
    vout_o[...] = _dot(dec, oW[...]) + ob[...]


def _tc_call(body, grid, in_specs, out_shapes, out_specs, name,
             interpret=False):
    return pl.pallas_call(
        body,
        grid=(grid,),
        in_specs=in_specs,
        out_specs=out_specs,
        out_shape=out_shapes,
        interpret=interpret,
        name=name,
    )


_MESHF = plsc.VectorSubcoreMesh
_CPF = pltpu.CompilerParams


@functools.cache
def _sc_gather_k(e_base, e_cnt):
    epw = e_cnt // NW
    mesh = _MESHF(core_axis_name="c", subcore_axis_name="s")

    @functools.partial(
        pl.kernel,
        out_type=[jax.ShapeDtypeStruct((e_cnt, HID), jnp.float32),
                  jax.ShapeDtypeStruct((e_cnt, HID), jnp.float32)],
        mesh=mesh,
        compiler_params=_CPF(use_tc_tiling_on_sc=False),
        scratch_types=[pltpu.VMEM((epw,), jnp.int32),
                       pltpu.VMEM((epw, HID), jnp.float32),
                       pltpu.SemaphoreType.DMA],
        name=f"sc_gather_{e_base}_{e_cnt}",
    )
    def sc_gather(ns_h, nd_h, src_h, dst_h, gs_h, gd_h, idx_v, rows_v, sem):
        wid = lax.axis_index("s") * 2 + lax.axis_index("c")
        lsl = pl.ds(wid * epw, epw)
        gsl = pl.ds(e_base + wid * epw, epw)
        pltpu.sync_copy(src_h.at[gsl], idx_v)
        pltpu.async_copy(ns_h.at[idx_v], rows_v, sem).wait()
        pltpu.sync_copy(rows_v, gs_h.at[lsl])
        pltpu.sync_copy(dst_h.at[gsl], idx_v)
        pltpu.async_copy(nd_h.at[idx_v], rows_v, sem).wait()
        pltpu.sync_copy(rows_v, gd_h.at[lsl])

    return sc_gather


@functools.cache
def _sc_count_k():
    mesh = _MESHF(core_axis_name="c", subcore_axis_name="s")

    @functools.partial(
        pl.kernel,
        out_type=[jax.ShapeDtypeStruct((2, N_NODES, HID), jnp.float32)],
        mesh=mesh,
        compiler_params=_CPF(use_tc_tiling_on_sc=False),
        scratch_types=[pltpu.VMEM((EPW,), jnp.int32),
                       pltpu.VMEM((EPW, HID), jnp.float32),
                       pltpu.VMEM_SHARED((N_NODES, HID), jnp.float32)],
        name="sc_count",
    )
    def sc_count(pa_h, dst_h, zeros_h, out_h, idx_v, pa_v, acc):
        sid = lax.axis_index("s")
        cid = lax.axis_index("c")
        base = (sid * 2 + cid) * EPW
        nsl = pl.ds(sid * NPT, NPT)
        pltpu.sync_copy(zeros_h.at[nsl], acc.at[nsl])
        plsc.subcore_barrier()
        pltpu.sync_copy(dst_h.at[pl.ds(base, EPW)], idx_v)
        pltpu.sync_copy(pa_h.at[pl.ds(base, EPW)], pa_v)
        pltpu.sync_copy(pa_v, acc.at[idx_v], add=True)
        plsc.subcore_barrier()
        pltpu.sync_copy(acc.at[nsl], out_h.at[cid, nsl])

    return sc_count


@functools.cache
def _sc_scatter_k(e_base, e_cnt):
    epw = e_cnt // NW
    nch = epw // CH
    mesh = _MESHF(core_axis_name="c", subcore_axis_name="s")

    @functools.partial(
        pl.kernel,
        out_type=[jax.ShapeDtypeStruct((2, N_NODES, LAT), jnp.float32)],
        mesh=mesh,
        compiler_params=_CPF(use_tc_tiling_on_sc=False),
        scratch_types=[pltpu.VMEM((nch, CH), jnp.int32),
                       pltpu.VMEM((CH, LAT), jnp.float32),
                       pltpu.VMEM_SHARED((N_NODES, LAT), jnp.float32)],
        name=f"sc_scatter_{e_base}_{e_cnt}",
    )
    def sc_scatter(pa_h, dst_h, zeros_h, out_h, idx2, pa_v, acc):
        sid = lax.axis_index("s")
        cid = lax.axis_index("c")
        lbase = (sid * 2 + cid) * epw
        gbase = e_base + lbase
        nsl = pl.ds(sid * NPT, NPT)
        pltpu.sync_copy(zeros_h.at[nsl], acc.at[nsl])
        plsc.subcore_barrier()
        for j in range(nch):
            pltpu.sync_copy(dst_h.at[pl.ds(gbase + j * CH, CH)], idx2.at[j])
            pltpu.sync_copy(pa_h.at[pl.ds(lbase + j * CH, CH)], pa_v)
            pltpu.sync_copy(pa_v, acc.at[idx2.at[j]], add=True)
        plsc.subcore_barrier()
        pltpu.sync_copy(acc.at[nsl], out_h.at[cid, nsl])

    return sc_scatter


CHUNKS = ((0, 96000), (96000, 64000))


def _forward_impl(vdata, edata, connectivity, params, interpret=False):
    f32 = jnp.float32
    src = connectivity[0]
    dst = connectivity[1]

    pce, pcn = params["core_edge"], params["core_node"]
    W1ce, W1cn = pce["W1"], pcn["W1"]
    A_e0, A_e = W1ce[0:LAT], W1ce[LAT:2 * LAT]
    A_v0s, A_vs = W1ce[2 * LAT:3 * LAT], W1ce[3 * LAT:4 * LAT]
    A_v0d, A_vd = W1ce[4 * LAT:5 * LAT], W1ce[5 * LAT:6 * LAT]
    B_v0, B_v, B_agg = W1cn[0:LAT], W1cn[LAT:2 * LAT], W1cn[2 * LAT:3 * LAT]

    def r2(x):
        return x.reshape(1, -1)

    en_, ee_ = params["enc_node"], params["enc_edge"]
    dn_, de_ = params["dec_node"], params["dec_edge"]

    w16 = _w_spec((LAT, HID))
    w1h = _w_spec((1, HID))
    w1l = _w_spec((1, LAT))

    ns, nd, pn, f_s, f_d, f_n = _tc_call(
        _enc_node_body, N_NODES // BN,
        in_specs=[_row_spec(BN, LAT), _w_spec((LAT, HID)), w1h,
                  _w_spec((HID, LAT)), w1l, w1l, w1l,
                  w16, w16, w16, w16, w16, w16],
        out_shapes=[jax.ShapeDtypeStruct((N_NODES, HID), f32)] * 6,
        out_specs=[_row_spec(BN, HID)] * 6,
        name="tc_enc_node", interpret=interpret,
    )(vdata, en_["W1"], r2(en_["b1"]), en_["W2"], r2(en_["b2"]),
      r2(en_["g"]), r2(en_["bt"]),
      A_vs, A_vd, B_v, A_v0s, A_v0d, B_v0)

    pe0, pec = _tc_call(
        _enc_edge_body, N_EDGES // BE,
        in_specs=[_row_spec(BE, HID), _w_spec((HID, HID)), w1h,
                  _w_spec((HID, LAT)), w1l, w1l, w1l, w16, w16],
        out_shapes=[jax.ShapeDtypeStruct((N_EDGES, HID), f32)] * 2,
        out_specs=[_row_spec(BE, HID)] * 2,
        name="tc_enc_edge", interpret=interpret,
    )(edata, ee_["W1"], r2(ee_["b1"]), ee_["W2"], r2(ee_["b2"]),
      r2(ee_["g"]), r2(ee_["bt"]), A_e0, A_e)

    zeros_nh = jnp.zeros((N_NODES, HID), f32)
    zeros_nl = jnp.zeros((N_NODES, LAT), f32)
    ones_e = jnp.ones((N_EDGES, HID), f32)

    if interpret:
        def do_gather(eb, ec, ns_, nd_):
            s_ = lax.dynamic_slice_in_dim(src, eb, ec)
            d_ = lax.dynamic_slice_in_dim(dst, eb, ec)
            return jnp.take(ns_, s_, axis=0), jnp.take(nd_, d_, axis=0)

        def do_count(x):
            s = jax.ops.segment_sum(x, dst, num_segments=N_NODES)
            return jnp.stack([s, jnp.zeros_like(s)])

        def do_scatter(eb, ec, x):
            d_ = lax.dynamic_slice_in_dim(dst, eb, ec)
            s = jax.ops.segment_sum(x, d_, num_segments=N_NODES)
            return jnp.stack([s, jnp.zeros_like(s)])
    else:
        def _unwrap(out):
            if isinstance(out, (list, tuple)):
                out = out[0]
            return out

        def do_gather(eb, ec, ns_, nd_):
            return _sc_gather_k(eb, ec)(ns_, nd_, src, dst)

        def do_count(x):
            return _unwrap(_sc_count_k()(x, dst, zeros_nh))

        def do_scatter(eb, ec, x):
            return _unwrap(_sc_scatter_k(eb, ec)(x, dst, zeros_nl))

    cntp = do_count(ones_e)

    def edge_specs(e_base, pec_is_half):
        off = e_base // BE
        full = pl.BlockSpec((BE, HID), lambda i, o=off: (i + o, 0))
        half = _row_spec(BE, HID)
        return [full, full if not pec_is_half else half, half, half]

    ew = [r2(pce["b1"]), pce["W2"], r2(pce["b2"]), r2(pce["g"]), r2(pce["bt"])]
    ew_specs = [w1h, _w_spec((HID, LAT)), w1l, w1l, w1l]

    pec_halves = None
    for t in range(CORE_STEPS - 1):
        gss, gds, ens, aggs = [], [], [], []
        for ci, (eb, ec) in enumerate(CHUNKS):
            gs, gd = do_gather(eb, ec, ns, nd)
            gss.append(gs)
            gds.append(gd)
        for ci, (eb, ec) in enumerate(CHUNKS):
            pec_in = pec if pec_halves is None else pec_halves[ci]
            en, pec_new = _tc_call(
                _edge_step_body, ec // BE,
                in_specs=edge_specs(eb, pec_halves is not None) + ew_specs
                + [w16],
                out_shapes=[jax.ShapeDtypeStruct((ec, LAT), f32),
                            jax.ShapeDtypeStruct((ec, HID), f32)],
                out_specs=[_row_spec(BE, LAT), _row_spec(BE, HID)],
                name=f"tc_edge_step{t}_{ci}", interpret=interpret,
            )(pe0, pec_in, gss[ci], gds[ci], *ew, A_e)
            ens.append(en)
            if pec_halves is None and ci == 0:
                pec_halves_new = [pec_new]
            elif ci == 0:
                pec_halves_new = [pec_new]
            else:
                pec_halves_new.append(pec_new)
            aggs.append(do_scatter(eb, ec, en))
        pec_halves = pec_halves_new

        ns, nd, pn = _tc_call(
            _node_step_body, N_NODES // BN,
            in_specs=[_row_spec(BN, HID), _row_spec(BN, LAT),
                      _row_spec(BN, LAT), _row_spec(BN, LAT),
                      _row_spec(BN, LAT), _row_spec(BN, HID),
                      _row_spec(BN, HID), _row_spec(BN, HID),
                      _row_spec(BN, HID), _row_spec(BN, HID),
                      w1h, _w_spec((HID, LAT)), w1l, w1l, w1l,
                      w16, w16, w16, w16],
            out_shapes=[jax.ShapeDtypeStruct((N_NODES, HID), f32)] * 3,
            out_specs=[_row_spec(BN, HID)] * 3,
            name=f"tc_node_step{t}", interpret=interpret,
        )(pn, aggs[0][0], aggs[0][1], aggs[1][0], aggs[1][1],
          cntp[0], cntp[1], f_s, f_d, f_n,
          r2(pcn["b1"]), pcn["W2"], r2(pcn["b2"]), r2(pcn["g"]),
          r2(pcn["bt"]), B_agg, A_vs, A_vd, B_v)

    gs3, gd3 = do_gather(0, N_EDGES, ns, nd)
    pec3 = jnp.concatenate(pec_halves, axis=0)
    en3 = _tc_call(
        _edge_core_body, N_EDGES // BE,
        in_specs=[_row_spec(BE, HID)] * 4 + ew_specs,
        out_shapes=jax.ShapeDtypeStruct((N_EDGES, LAT), f32),
        out_specs=_row_spec(BE, LAT),
        name="tc_edge_core_last", interpret=interpret,
    )(pe0, pec3, gs3, gd3, *ew)

    aggp = do_scatter(0, N_EDGES, en3)

    e_out = _tc_call(
        _edge_dec_body, N_EDGES // BE,
        in_specs=[_row_spec(BE, LAT),
                  _w_spec((LAT, HID)), w1h, _w_spec((HID, LAT)),
                  w1l, w1l, w1l, _w_spec((LAT, LAT)), w1l],
        out_shapes=jax.ShapeDtypeStruct((N_EDGES, LAT), f32),
        out_specs=_row_spec(BE, LAT),
        name="tc_edge_dec", interpret=interpret,
    )(en3, de_["W1"], r2(de_["b1"]), de_["W2"], r2(de_["b2"]),
      r2(de_["g"]), r2(de_["bt"]),
      params["dec_edge_out_W"], r2(params["dec_edge_out_b"]))

    v_out = _tc_call(
        _node_last_body, N_NODES // BN,
        in_specs=[_row_spec(BN, HID), _row_spec(BN, LAT),
                  _row_spec(BN, LAT), _row_spec(BN, HID),
                  _row_spec(BN, HID),
                  w1h, _w_spec((HID, LAT)), w1l, w1l, w1l, w16,
                  _w_spec((LAT, HID)), w1h, _w_spec((HID, LAT)),
                  w1l, w1l, w1l, _w_spec((LAT, LAT)), w1l],
        out_shapes=jax.ShapeDtypeStruct((N_NODES, LAT), f32),
        out_specs=_row_spec(BN, LAT),
        name="tc_node_last", interpret=interpret,
    )(pn, aggp[0], aggp[1], cntp[0], cntp[1],
      r2(pcn["b1"]), pcn["W2"], r2(pcn["b2"]), r2(pcn["g"]),
      r2(pcn["bt"]), B_agg,
      dn_["W1"], r2(dn_["b1"]), dn_["W2"], r2(dn_["b2"]),
      r2(dn_["g"]), r2(dn_["bt"]),
      params["dec_node_out_W"], r2(params["dec_node_out_b"]))

    return (v_out, e_out)


def kernel(vdata, edata, connectivity, cdata, metadata, params):
    return _forward_impl(vdata, edata, connectivity, params)


# speedup vs baseline: 1.1741x; 1.1741x over previous
import functools
import jax
import jax.numpy as jnp
from jax import lax
from jax.experimental import pallas as pl
from jax.experimental.pallas import tpu as pltpu
from jax.experimental.pallas import tpu_sc as plsc

N_NODES = 10000
N_EDGES = 160000
LAT = 128
HID = 16
CORE_STEPS = 3

BN = 1000
BE = 2000
NW = 32
EPW = N_EDGES // NW
NPT = N_NODES // 16
CH = 200
NCH = EPW // CH

_EPS = 1e-5


def _ln(h, g, bt):
    mu = jnp.mean(h, axis=-1, keepdims=True)
    var = jnp.mean((h - mu) * (h - mu), axis=-1, keepdims=True)
    return (h - mu) * lax.rsqrt(var + _EPS) * g + bt


def _dot(a, b):
    return jnp.dot(a, b, preferred_element_type=jnp.float32)


def _mlp_tail(h1pre, W2, b2, g, bt):
    h = jnp.maximum(h1pre, 0.0)
    h = jnp.maximum(_dot(h, W2) + b2, 0.0)
    return _ln(h, g, bt)


def _row_spec(b, d):
    return pl.BlockSpec((b, d), lambda i: (i, 0))


def _w_spec(shape):
    return pl.BlockSpec(shape, lambda i: tuple(0 for _ in shape))


def _enc_node_body(x, W1, b1, W2, b2, g, bt, Avs, Avd, Bv, Av0s, Av0d, Bv0,
                   ns_o, nd_o, pn_o, fs_o, fd_o, fn_o):
    h1 = _dot(x[...], W1[...]) + b1[...]
    v0 = _mlp_tail(h1, W2[...], b2[...], g[...], bt[...])
    fs = _dot(v0, Av0s[...])
    fd = _dot(v0, Av0d[...])
    fn = _dot(v0, Bv0[...])
    fs_o[...] = fs
    fd_o[...] = fd
    fn_o[...] = fn
    ns_o[...] = fs + _dot(v0, Avs[...])
    nd_o[...] = fd + _dot(v0, Avd[...])
    pn_o[...] = fn + _dot(v0, Bv[...])


def _enc_edge_body(x, W1, b1, W2, b2, g, bt, A0, A1, pe0_o, pec_o):
    h1 = _dot(x[...], W1[...]) + b1[...]
    e0 = _mlp_tail(h1, W2[...], b2[...], g[...], bt[...])
    pe0_o[...] = _dot(e0, A0[...])
    pec_o[...] = _dot(e0, A1[...])


def _edge_step_body(pe0, pec, hs, b1, W2, b2, g, bt, Ae,
                    en_o, pec_o):
    h1 = pe0[...] + pec[...] + hs[...] + b1[...]
    en = _mlp_tail(h1, W2[...], b2[...], g[...], bt[...])
    en_o[...] = en
    pec_o[...] = _dot(en, Ae[...])


def _edge_core_body(pe0, pec, hs0, hs1, b1, W2, b2, g, bt, en_o):
    i = pl.program_id(0)
    nb0 = CHUNKS[0][1] // BE
    hs = jnp.where(i < nb0, hs0[...], hs1[...])
    h1 = pe0[...] + pec[...] + hs + b1[...]
    en_o[...] = _mlp_tail(h1, W2[...], b2[...], g[...], bt[...])


def _edge_dec_body(en, dW1, db1, dW2, db2, dg, dbt, oW, ob, eout_o):
    d1 = _dot(en[...], dW1[...]) + db1[...]
    dec = _mlp_tail(d1, dW2[...], db2[...], dg[...], dbt[...])
    eout_o[...] = _dot(dec, oW[...]) + ob[...]


def _node_step_body(pn, a0, a1, a2, a3, c0, c1, fs, fd, fn, b1, W2, b2, g, bt,
                    Bagg, Avs, Avd, Bv, ns_o, nd_o, pn_o):
    cm = jnp.maximum(c0[...] + c1[...], 1.0)[:, 0:1]
    agg = (a0[...] + a1[...] + a2[...] + a3[...]) / cm
    h1 = pn[...] + _dot(agg, Bagg[...]) + b1[...]
    vn = _mlp_tail(h1, W2[...], b2[...], g[...], bt[...])
    ns_o[...] = fs[...] + _dot(vn, Avs[...])
    nd_o[...] = fd[...] + _dot(vn, Avd[...])
    pn_o[...] = fn[...] + _dot(vn, Bv[...])


def _node_last_body(pn, a0, a1, a2, a3, c0, c1, b1, W2, b2, g, bt, Bagg,
                    dW1, db1, dW2, db2, dg, dbt, oW, ob, vout_o):
    cm = jnp.maximum(c0[...] + c1[...], 1.0)[:, 0:1]
    agg = (a0[...] + a1[...] + a2[...] + a3[...]) / cm
    h1 = pn[...] + _dot(agg, Bagg[...]) + b1[...]
    vn = _mlp_tail(h1, W2[...], b2[...], g[...], bt[...])
    d1 = _dot(vn, dW1[...]) + db1[...]
    dec = _mlp_tail(d1, dW2[...], db2[...], dg[...], dbt[...])
    vout_o[...] = _dot(dec, oW[...]) + ob[...]


def _tc_call(body, grid, in_specs, out_shapes, out_specs, name,
             interpret=False):
    return pl.pallas_call(
        body,
        grid=(grid,),
        in_specs=in_specs,
        out_specs=out_specs,
        out_shape=out_shapes,
        interpret=interpret,
        name=name,
    )


_MESHF = plsc.VectorSubcoreMesh
_CPF = pltpu.CompilerParams


@functools.cache
def _sc_gather_k(e_base, e_cnt):
    epw = e_cnt // NW
    mesh = _MESHF(core_axis_name="c", subcore_axis_name="s")

    @functools.partial(
        pl.kernel,
        out_type=[jax.ShapeDtypeStruct((e_cnt, HID), jnp.float32)],
        mesh=mesh,
        compiler_params=_CPF(use_tc_tiling_on_sc=False),
        scratch_types=[pltpu.VMEM((epw,), jnp.int32),
                       pltpu.VMEM((epw,), jnp.int32),
                       pltpu.VMEM((epw, HID), jnp.float32),
                       pltpu.VMEM((epw, HID), jnp.float32),
                       pltpu.SemaphoreType.DMA],
        name=f"sc_gather_{e_base}_{e_cnt}",
    )
    def sc_gather(ns_h, nd_h, src_h, dst_h, hs_h, idx_s, idx_d, rs_v, rd_v,
                  sem):
        wid = lax.axis_index("s") * 2 + lax.axis_index("c")
        lsl = pl.ds(wid * epw, epw)
        gsl = pl.ds(e_base + wid * epw, epw)
        pltpu.sync_copy(src_h.at[gsl], idx_s)
        pltpu.sync_copy(dst_h.at[gsl], idx_d)
        cps = pltpu.async_copy(ns_h.at[idx_s], rs_v, sem)
        cpd = pltpu.async_copy(nd_h.at[idx_d], rd_v, sem)
        cps.wait()
        cpd.wait()

        def add4(i, _):
            for k in range(4):
                r = i * 4 + k
                rs_v[r, :] = rs_v[r, :] + rd_v[r, :]
            return 0

        lax.fori_loop(0, epw // 4, add4, 0, unroll=False)
        pltpu.sync_copy(rs_v, hs_h.at[lsl])

    return sc_gather


@functools.cache
def _sc_count_k():
    mesh = _MESHF(core_axis_name="c", subcore_axis_name="s")

    @functools.partial(
        pl.kernel,
        out_type=[jax.ShapeDtypeStruct((N_NODES, HID), jnp.float32),
                  jax.ShapeDtypeStruct((N_NODES, HID), jnp.float32)],
        mesh=mesh,
        compiler_params=_CPF(use_tc_tiling_on_sc=False),
        scratch_types=[pltpu.VMEM((EPW,), jnp.int32),
                       pltpu.VMEM((EPW, HID), jnp.float32),
                       pltpu.VMEM_SHARED((N_NODES, HID), jnp.float32)],
        name="sc_count",
    )
    def sc_count(dst_h, zeros_h, out0_h, out1_h, idx_v, pa_v, acc):
        sid = lax.axis_index("s")
        cid = lax.axis_index("c")
        base = (sid * 2 + cid) * EPW
        nsl = pl.ds(sid * NPT, NPT)
        pltpu.sync_copy(zeros_h.at[nsl], acc.at[nsl])

        one = jnp.ones((HID,), jnp.float32)

        def fill4(i, _):
            for k in range(4):
                pa_v[i * 4 + k, :] = one
            return 0

        lax.fori_loop(0, EPW // 4, fill4, 0, unroll=False)
        plsc.subcore_barrier()
        pltpu.sync_copy(dst_h.at[pl.ds(base, EPW)], idx_v)
        pltpu.sync_copy(pa_v, acc.at[idx_v], add=True)
        plsc.subcore_barrier()

        @pl.when(cid == 0)
        def _():
            pltpu.sync_copy(acc.at[nsl], out0_h.at[nsl])

        @pl.when(cid == 1)
        def _():
            pltpu.sync_copy(acc.at[nsl], out1_h.at[nsl])

    return sc_count


@functools.cache
def _sc_scatter_k(e_base, e_cnt, pa_base):
    epw = e_cnt // NW
    nch = epw // CH
    mesh = _MESHF(core_axis_name="c", subcore_axis_name="s")

    @functools.partial(
        pl.kernel,
        out_type=[jax.ShapeDtypeStruct((N_NODES, LAT), jnp.float32),
                  jax.ShapeDtypeStruct((N_NODES, LAT), jnp.float32)],
        mesh=mesh,
        compiler_params=_CPF(use_tc_tiling_on_sc=False),
        scratch_types=[pltpu.VMEM((nch, CH), jnp.int32),
                       pltpu.VMEM((CH, LAT), jnp.float32),
                       pltpu.VMEM_SHARED((N_NODES, LAT), jnp.float32)],
        name=f"sc_scatter_{e_base}_{e_cnt}",
    )
    def sc_scatter(pa_h, dst_h, zeros_h, out0_h, out1_h, idx2, pa_v, acc):
        sid = lax.axis_index("s")
        cid = lax.axis_index("c")
        lbase = (sid * 2 + cid) * epw
        gbase = e_base + lbase
        pbase = pa_base + lbase
        nsl = pl.ds(sid * NPT, NPT)
        pltpu.sync_copy(zeros_h.at[nsl], acc.at[nsl])
        plsc.subcore_barrier()
        for j in range(nch):
            pltpu.sync_copy(dst_h.at[pl.ds(gbase + j * CH, CH)], idx2.at[j])
            pltpu.sync_copy(pa_h.at[pl.ds(pbase + j * CH, CH)], pa_v)
            pltpu.sync_copy(pa_v, acc.at[idx2.at[j]], add=True)
        plsc.subcore_barrier()

        @pl.when(cid == 0)
        def _():
            pltpu.sync_copy(acc.at[nsl], out0_h.at[nsl])

        @pl.when(cid == 1)
        def _():
            pltpu.sync_copy(acc.at[nsl], out1_h.at[nsl])

    return sc_scatter


CHUNKS = ((0, 96000), (96000, 64000))


def _forward_impl(vdata, edata, connectivity, params, interpret=False):
    f32 = jnp.float32
    src = connectivity[0]
    dst = connectivity[1]

    pce, pcn = params["core_edge"], params["core_node"]
    W1ce, W1cn = pce["W1"], pcn["W1"]
    A_e0, A_e = W1ce[0:LAT], W1ce[LAT:2 * LAT]
    A_v0s, A_vs = W1ce[2 * LAT:3 * LAT], W1ce[3 * LAT:4 * LAT]
    A_v0d, A_vd = W1ce[4 * LAT:5 * LAT], W1ce[5 * LAT:6 * LAT]
    B_v0, B_v, B_agg = W1cn[0:LAT], W1cn[LAT:2 * LAT], W1cn[2 * LAT:3 * LAT]

    def r2(x):
        return x.reshape(1, -1)

    en_, ee_ = params["enc_node"], params["enc_edge"]
    dn_, de_ = params["dec_node"], params["dec_edge"]

    w16 = _w_spec((LAT, HID))
    w1h = _w_spec((1, HID))
    w1l = _w_spec((1, LAT))

    ns, nd, pn, f_s, f_d, f_n = _tc_call(
        _enc_node_body, N_NODES // BN,
        in_specs=[_row_spec(BN, LAT), _w_spec((LAT, HID)), w1h,
                  _w_spec((HID, LAT)), w1l, w1l, w1l,
                  w16, w16, w16, w16, w16, w16],
        out_shapes=[jax.ShapeDtypeStruct((N_NODES, HID), f32)] * 6,
        out_specs=[_row_spec(BN, HID)] * 6,
        name="tc_enc_node", interpret=interpret,
    )(vdata, en_["W1"], r2(en_["b1"]), en_["W2"], r2(en_["b2"]),
      r2(en_["g"]), r2(en_["bt"]),
      A_vs, A_vd, B_v, A_v0s, A_v0d, B_v0)

    pe0, pec = _tc_call(
        _enc_edge_body, N_EDGES // BE,
        in_specs=[_row_spec(BE, HID), _w_spec((HID, HID)), w1h,
                  _w_spec((HID, LAT)), w1l, w1l, w1l, w16, w16],
        out_shapes=[jax.ShapeDtypeStruct((N_EDGES, HID), f32)] * 2,
        out_specs=[_row_spec(BE, HID)] * 2,
        name="tc_enc_edge", interpret=interpret,
    )(edata, ee_["W1"], r2(ee_["b1"]), ee_["W2"], r2(ee_["b2"]),
      r2(ee_["g"]), r2(ee_["bt"]), A_e0, A_e)

    zeros_nh = jnp.zeros((N_NODES, HID), f32)
    zeros_nl = jnp.zeros((N_NODES, LAT), f32)

    if interpret:
        def do_gather(eb, ec, ns_, nd_):
            s_ = lax.dynamic_slice_in_dim(src, eb, ec)
            d_ = lax.dynamic_slice_in_dim(dst, eb, ec)
            return jnp.take(ns_, s_, axis=0) + jnp.take(nd_, d_, axis=0)

        def do_count():
            s = jax.ops.segment_sum(jnp.ones((N_EDGES, HID), f32), dst,
                                    num_segments=N_NODES)
            return s, jnp.zeros_like(s)

        def do_scatter(eb, ec, pb, x):
            d_ = lax.dynamic_slice_in_dim(dst, eb, ec)
            x_ = lax.dynamic_slice_in_dim(x, pb, ec) if x.shape[0] != ec else x
            s = jax.ops.segment_sum(x_, d_, num_segments=N_NODES)
            return s, jnp.zeros_like(s)
    else:
        def do_gather(eb, ec, ns_, nd_):
            out = _sc_gather_k(eb, ec)(ns_, nd_, src, dst)
            return out[0] if isinstance(out, (list, tuple)) else out

        def do_count():
            return _sc_count_k()(dst, zeros_nh)

        def do_scatter(eb, ec, pb, x):
            return _sc_scatter_k(eb, ec, pb)(x, dst, zeros_nl)

    cntp = do_count()

    def edge_specs(e_base, pec_is_half):
        off = e_base // BE
        full = pl.BlockSpec((BE, HID), lambda i, o=off: (i + o, 0))
        half = _row_spec(BE, HID)
        return [full, half if pec_is_half else full, half]

    ew = [r2(pce["b1"]), pce["W2"], r2(pce["b2"]), r2(pce["g"]), r2(pce["bt"])]
    ew_specs = [w1h, _w_spec((HID, LAT)), w1l, w1l, w1l]

    pec_halves = None
    for t in range(CORE_STEPS - 1):
        hsums = [do_gather(eb, ec, ns, nd) for (eb, ec) in CHUNKS]
        ens, aggs, pec_new = [], [], []
        for ci, (eb, ec) in enumerate(CHUNKS):
            pec_in = pec if pec_halves is None else pec_halves[ci]
            en, pc = _tc_call(
                _edge_step_body, ec // BE,
                in_specs=edge_specs(eb, pec_halves is not None) + ew_specs
                + [w16],
                out_shapes=[jax.ShapeDtypeStruct((ec, LAT), f32),
                            jax.ShapeDtypeStruct((ec, HID), f32)],
                out_specs=[_row_spec(BE, LAT), _row_spec(BE, HID)],
                name=f"tc_edge_step{t}_{ci}", interpret=interpret,
            )(pe0, pec_in, hsums[ci], *ew, A_e)
            ens.append(en)
            pec_new.append(pc)
            aggs.append(do_scatter(eb, ec, 0, en))
        pec_halves = pec_new

        ns, nd, pn = _tc_call(
            _node_step_body, N_NODES // BN,
            in_specs=[_row_spec(BN, HID), _row_spec(BN, LAT),
                      _row_spec(BN, LAT), _row_spec(BN, LAT),
                      _row_spec(BN, LAT), _row_spec(BN, HID),
                      _row_spec(BN, HID), _row_spec(BN, HID),
                      _row_spec(BN, HID), _row_spec(BN, HID),
                      w1h, _w_spec((HID, LAT)), w1l, w1l, w1l,
                      w16, w16, w16, w16],
            out_shapes=[jax.ShapeDtypeStruct((N_NODES, HID), f32)] * 3,
            out_specs=[_row_spec(BN, HID)] * 3,
            name=f"tc_node_step{t}", interpret=interpret,
        )(pn, aggs[0][0], aggs[0][1], aggs[1][0], aggs[1][1],
          cntp[0], cntp[1], f_s, f_d, f_n,
          r2(pcn["b1"]), pcn["W2"], r2(pcn["b2"]), r2(pcn["g"]),
          r2(pcn["bt"]), B_agg, A_vs, A_vd, B_v)

    hs3 = [do_gather(eb, ec, ns, nd) for (eb, ec) in CHUNKS]
    pec3 = jnp.concatenate(pec_halves, axis=0)
    nb0 = CHUNKS[0][1] // BE
    nb1 = CHUNKS[1][1] // BE
    en3 = _tc_call(
        _edge_core_body, N_EDGES // BE,
        in_specs=[_row_spec(BE, HID), _row_spec(BE, HID),
                  pl.BlockSpec((BE, HID),
                               lambda i: (jnp.minimum(i, nb0 - 1), 0)),
                  pl.BlockSpec((BE, HID),
                               lambda i: (jnp.maximum(i - nb0, 0), 0))]
        + ew_specs,
        out_shapes=jax.ShapeDtypeStruct((N_EDGES, LAT), f32),
        out_specs=_row_spec(BE, LAT),
        name="tc_edge_core_last", interpret=interpret,
    )(pe0, pec3, hs3[0], hs3[1], *ew)

    aggs3 = [do_scatter(eb, ec, eb, en3) for (eb, ec) in CHUNKS]

    e_out = _tc_call(
        _edge_dec_body, N_EDGES // BE,
        in_specs=[_row_spec(BE, LAT),
                  _w_spec((LAT, HID)), w1h, _w_spec((HID, LAT)),
                  w1l, w1l, w1l, _w_spec((LAT, LAT)), w1l],
        out_shapes=jax.ShapeDtypeStruct((N_EDGES, LAT), f32),
        out_specs=_row_spec(BE, LAT),
        name="tc_edge_dec", interpret=interpret,
    )(en3, de_["W1"], r2(de_["b1"]), de_["W2"], r2(de_["b2"]),
      r2(de_["g"]), r2(de_["bt"]),
      params["dec_edge_out_W"], r2(params["dec_edge_out_b"]))

    v_out = _tc_call(
        _node_last_body, N_NODES // BN,
        in_specs=[_row_spec(BN, HID), _row_spec(BN, LAT),
                  _row_spec(BN, LAT), _row_spec(BN, LAT),
                  _row_spec(BN, LAT), _row_spec(BN, HID),
                  _row_spec(BN, HID),
                  w1h, _w_spec((HID, LAT)), w1l, w1l, w1l, w16,
                  _w_spec((LAT, HID)), w1h, _w_spec((HID, LAT)),
                  w1l, w1l, w1l, _w_spec((LAT, LAT)), w1l],
        out_shapes=jax.ShapeDtypeStruct((N_NODES, LAT), f32),
        out_specs=_row_spec(BN, LAT),
        name="tc_node_last", interpret=interpret,
    )(pn, aggs3[0][0], aggs3[0][1], aggs3[1][0], aggs3[1][1],
      cntp[0], cntp[1],
      r2(pcn["b1"]), pcn["W2"], r2(pcn["b2"]), r2(pcn["g"]),
      r2(pcn["bt"]), B_agg,
      dn_["W1"], r2(dn_["b1"]), dn_["W2"], r2(dn_["b2"]),
      r2(dn_["g"]), r2(dn_["bt"]),
      params["dec_node_out_W"], r2(params["dec_node_out_b"]))

    return (v_out, e_out)


def kernel(vdata, edata, connectivity, cdata, metadata, params):
    return _forward_impl(vdata, edata, connectivity, params)


# speedup vs baseline: 1.6754x; 1.4270x over previous
import functools
import jax
import jax.numpy as jnp
from jax import lax
from jax.experimental import pallas as pl
from jax.experimental.pallas import tpu as pltpu
from jax.experimental.pallas import tpu_sc as plsc

N_NODES = 10000
N_EDGES = 160000
LAT = 128
HID = 16
CORE_STEPS = 3

BN = 1000
BE = 3200
BP = BE // 8
EP8 = N_EDGES // 8
NW = 32
EPW = N_EDGES // NW
NPT = N_NODES // 16
CH = 200
NCH = EPW // CH

_EPS = 1e-5


def _ln(h, g, bt):
    mu = jnp.mean(h, axis=-1, keepdims=True)
    var = jnp.mean((h - mu) * (h - mu), axis=-1, keepdims=True)
    return (h - mu) * lax.rsqrt(var + _EPS) * g + bt


def _dot(a, b):
    return jnp.dot(a, b, preferred_element_type=jnp.float32)


def _mlp_tail(h1pre, W2, b2, g, bt):
    h = jnp.maximum(h1pre, 0.0)
    h = jnp.maximum(_dot(h, W2) + b2, 0.0)
    return _ln(h, g, bt)


def _row_spec(b, d):
    return pl.BlockSpec((b, d), lambda i: (i, 0))


def _w_spec(shape):
    return pl.BlockSpec(shape, lambda i: tuple(0 for _ in shape))


def _enc_node_body(x, W1, b1, W2, b2, g, bt, Avs, Avd, Bv, Av0s, Av0d, Bv0,
                   ns_o, nd_o, pn_o, fs_o, fd_o, fn_o):
    h1 = _dot(x[...], W1[...]) + b1[...]
    v0 = _mlp_tail(h1, W2[...], b2[...], g[...], bt[...])
    fs = _dot(v0, Av0s[...])
    fd = _dot(v0, Av0d[...])
    fn = _dot(v0, Bv0[...])
    fs_o[...] = fs
    fd_o[...] = fd
    fn_o[...] = fn
    ns_o[...] = fs + _dot(v0, Avs[...])
    nd_o[...] = fd + _dot(v0, Avd[...])
    pn_o[...] = fn + _dot(v0, Bv[...])


def _enc_edge_body(xp, W1b, b1t, W2b, b2t, g3, bt3, A0b, A1b, pe0_o, pec_o):
    h1p = jnp.maximum(_dot(xp[...], W1b[...]) + b1t[...], 0.0)
    enw = jnp.maximum(_dot(h1p, W2b[...]) + b2t[...], 0.0)
    e3 = jnp.reshape(enw, (BP, 8, LAT))
    mu = jnp.mean(e3, axis=-1, keepdims=True)
    var = jnp.mean((e3 - mu) * (e3 - mu), axis=-1, keepdims=True)
    e3 = (e3 - mu) * lax.rsqrt(var + _EPS) * g3[...] + bt3[...]
    ef = jnp.reshape(e3, (BP, 8 * LAT))
    pe0_o[...] = _dot(ef, A0b[...])
    pec_o[...] = _dot(ef, A1b[...])


def _edge_step_body(pe0, pec, hs, b1t, W2b, b2t, g3, bt3, Aeb,
                    en_o, pec_o):
    h1p = jnp.maximum(pe0[...] + pec[...] + hs[...] + b1t[...], 0.0)
    enw = jnp.maximum(_dot(h1p, W2b[...]) + b2t[...], 0.0)
    e3 = jnp.reshape(enw, (BP, 8, LAT))
    mu = jnp.mean(e3, axis=-1, keepdims=True)
    var = jnp.mean((e3 - mu) * (e3 - mu), axis=-1, keepdims=True)
    e3 = (e3 - mu) * lax.rsqrt(var + _EPS) * g3[...] + bt3[...]
    en_o[...] = jnp.reshape(e3, (BE, LAT))
    pec_o[...] = _dot(jnp.reshape(e3, (BP, 8 * LAT)), Aeb[...])


def _edge_core_body(pe0, pec, hs0, hs1, b1t, W2b, b2t, g3, bt3, en_o):
    i = pl.program_id(0)
    nb0 = CHUNKS[0][1] // BE
    hs = jnp.where(i < nb0, hs0[...], hs1[...])
    h1p = jnp.maximum(pe0[...] + pec[...] + hs + b1t[...], 0.0)
    enw = jnp.maximum(_dot(h1p, W2b[...]) + b2t[...], 0.0)
    e3 = jnp.reshape(enw, (BP, 8, LAT))
    mu = jnp.mean(e3, axis=-1, keepdims=True)
    var = jnp.mean((e3 - mu) * (e3 - mu), axis=-1, keepdims=True)
    e3 = (e3 - mu) * lax.rsqrt(var + _EPS) * g3[...] + bt3[...]
    en_o[...] = jnp.reshape(e3, (BE, LAT))


def _edge_dec_body(en, dW1, db1, dW2, db2, dg, dbt, oW, ob, eout_o):
    d1 = _dot(en[...], dW1[...]) + db1[...]
    dec = _mlp_tail(d1, dW2[...], db2[...], dg[...], dbt[...])
    eout_o[...] = _dot(dec, oW[...]) + ob[...]


def _node_step_body(pn, a0, a1, a2, a3, c0, c1, fs, fd, fn, b1, W2, b2, g, bt,
                    Bagg, Avs, Avd, Bv, ns_o, nd_o, pn_o):
    cm = jnp.maximum(c0[...] + c1[...], 1.0)[:, 0:1]
    agg = (a0[...] + a1[...] + a2[...] + a3[...]) / cm
    h1 = pn[...] + _dot(agg, Bagg[...]) + b1[...]
    vn = _mlp_tail(h1, W2[...], b2[...], g[...], bt[...])
    ns_o[...] = fs[...] + _dot(vn, Avs[...])
    nd_o[...] = fd[...] + _dot(vn, Avd[...])
    pn_o[...] = fn[...] + _dot(vn, Bv[...])


def _node_last_body(pn, a0, a1, a2, a3, c0, c1, b1, W2, b2, g, bt, Bagg,
                    dW1, db1, dW2, db2, dg, dbt, oW, ob, vout_o):
    cm = jnp.maximum(c0[...] + c1[...], 1.0)[:, 0:1]
    agg = (a0[...] + a1[...] + a2[...] + a3[...]) / cm
    h1 = pn[...] + _dot(agg, Bagg[...]) + b1[...]
    vn = _mlp_tail(h1, W2[...], b2[...], g[...], bt[...])
    d1 = _dot(vn, dW1[...]) + db1[...]
    dec = _mlp_tail(d1, dW2[...], db2[...], dg[...], dbt[...])
    vout_o[...] = _dot(dec, oW[...]) + ob[...]


def _tc_call(body, grid, in_specs, out_shapes, out_specs, name,
             interpret=False):
    return pl.pallas_call(
        body,
        grid=(grid,),
        in_specs=in_specs,
        out_specs=out_specs,
        out_shape=out_shapes,
        interpret=interpret,
        name=name,
    )


_MESHF = plsc.VectorSubcoreMesh
_CPF = pltpu.CompilerParams


@functools.cache
def _sc_gather_k(e_base, e_cnt):
    epw = e_cnt // NW
    mesh = _MESHF(core_axis_name="c", subcore_axis_name="s")

    @functools.partial(
        pl.kernel,
        out_type=[jax.ShapeDtypeStruct((e_cnt, HID), jnp.float32)],
        mesh=mesh,
        compiler_params=_CPF(use_tc_tiling_on_sc=False),
        scratch_types=[pltpu.VMEM((epw,), jnp.int32),
                       pltpu.VMEM((epw,), jnp.int32),
                       pltpu.VMEM((epw, HID), jnp.float32),
                       pltpu.VMEM((epw, HID), jnp.float32),
                       pltpu.SemaphoreType.DMA],
        name=f"sc_gather_{e_base}_{e_cnt}",
    )
    def sc_gather(ns_h, nd_h, src_h, dst_h, hs_h, idx_s, idx_d, rs_v, rd_v,
                  sem):
        wid = lax.axis_index("s") * 2 + lax.axis_index("c")
        lsl = pl.ds(wid * epw, epw)
        gsl = pl.ds(e_base + wid * epw, epw)
        pltpu.sync_copy(src_h.at[gsl], idx_s)
        pltpu.sync_copy(dst_h.at[gsl], idx_d)
        cps = pltpu.async_copy(ns_h.at[idx_s], rs_v, sem)
        cpd = pltpu.async_copy(nd_h.at[idx_d], rd_v, sem)
        cps.wait()
        cpd.wait()

        def add4(i, _):
            for k in range(4):
                r = i * 4 + k
                rs_v[r, :] = rs_v[r, :] + rd_v[r, :]
            return 0

        lax.fori_loop(0, epw // 4, add4, 0, unroll=False)
        pltpu.sync_copy(rs_v, hs_h.at[lsl])

    return sc_gather


@functools.cache
def _sc_count_k():
    mesh = _MESHF(core_axis_name="c", subcore_axis_name="s")

    @functools.partial(
        pl.kernel,
        out_type=[jax.ShapeDtypeStruct((N_NODES, HID), jnp.float32),
                  jax.ShapeDtypeStruct((N_NODES, HID), jnp.float32)],
        mesh=mesh,
        compiler_params=_CPF(use_tc_tiling_on_sc=False),
        scratch_types=[pltpu.VMEM((EPW,), jnp.int32),
                       pltpu.VMEM((EPW, HID), jnp.float32),
                       pltpu.VMEM_SHARED((N_NODES, HID), jnp.float32)],
        name="sc_count",
    )
    def sc_count(dst_h, zeros_h, out0_h, out1_h, idx_v, pa_v, acc):
        sid = lax.axis_index("s")
        cid = lax.axis_index("c")
        base = (sid * 2 + cid) * EPW
        nsl = pl.ds(sid * NPT, NPT)
        pltpu.sync_copy(zeros_h.at[nsl], acc.at[nsl])

        one = jnp.ones((HID,), jnp.float32)

        def fill4(i, _):
            for k in range(4):
                pa_v[i * 4 + k, :] = one
            return 0

        lax.fori_loop(0, EPW // 4, fill4, 0, unroll=False)
        plsc.subcore_barrier()
        pltpu.sync_copy(dst_h.at[pl.ds(base, EPW)], idx_v)
        pltpu.sync_copy(pa_v, acc.at[idx_v], add=True)
        plsc.subcore_barrier()

        @pl.when(cid == 0)
        def _():
            pltpu.sync_copy(acc.at[nsl], out0_h.at[nsl])

        @pl.when(cid == 1)
        def _():
            pltpu.sync_copy(acc.at[nsl], out1_h.at[nsl])

    return sc_count


@functools.cache
def _sc_scatter_k(e_base, e_cnt, pa_base):
    epw = e_cnt // NW
    nch = epw // CH
    mesh = _MESHF(core_axis_name="c", subcore_axis_name="s")

    @functools.partial(
        pl.kernel,
        out_type=[jax.ShapeDtypeStruct((N_NODES, LAT), jnp.float32),
                  jax.ShapeDtypeStruct((N_NODES, LAT), jnp.float32)],
        mesh=mesh,
        compiler_params=_CPF(use_tc_tiling_on_sc=False),
        scratch_types=[pltpu.VMEM((nch, CH), jnp.int32),
                       pltpu.VMEM((CH, LAT), jnp.float32),
                       pltpu.VMEM_SHARED((N_NODES, LAT), jnp.float32)],
        name=f"sc_scatter_{e_base}_{e_cnt}",
    )
    def sc_scatter(pa_h, dst_h, zeros_h, out0_h, out1_h, idx2, pa_v, acc):
        sid = lax.axis_index("s")
        cid = lax.axis_index("c")
        lbase = (sid * 2 + cid) * epw
        gbase = e_base + lbase
        pbase = pa_base + lbase
        nsl = pl.ds(sid * NPT, NPT)
        pltpu.sync_copy(zeros_h.at[nsl], acc.at[nsl])
        plsc.subcore_barrier()
        for j in range(nch):
            pltpu.sync_copy(dst_h.at[pl.ds(gbase + j * CH, CH)], idx2.at[j])
            pltpu.sync_copy(pa_h.at[pl.ds(pbase + j * CH, CH)], pa_v)
            pltpu.sync_copy(pa_v, acc.at[idx2.at[j]], add=True)
        plsc.subcore_barrier()

        @pl.when(cid == 0)
        def _():
            pltpu.sync_copy(acc.at[nsl], out0_h.at[nsl])

        @pl.when(cid == 1)
        def _():
            pltpu.sync_copy(acc.at[nsl], out1_h.at[nsl])

    return sc_scatter


CHUNKS = ((0, 96000), (96000, 64000))


def _forward_impl(vdata, edata, connectivity, params, interpret=False):
    f32 = jnp.float32
    src = connectivity[0]
    dst = connectivity[1]

    pce, pcn = params["core_edge"], params["core_node"]
    W1ce, W1cn = pce["W1"], pcn["W1"]
    A_e0, A_e = W1ce[0:LAT], W1ce[LAT:2 * LAT]
    A_v0s, A_vs = W1ce[2 * LAT:3 * LAT], W1ce[3 * LAT:4 * LAT]
    A_v0d, A_vd = W1ce[4 * LAT:5 * LAT], W1ce[5 * LAT:6 * LAT]
    B_v0, B_v, B_agg = W1cn[0:LAT], W1cn[LAT:2 * LAT], W1cn[2 * LAT:3 * LAT]

    def r2(x):
        return x.reshape(1, -1)

    en_, ee_ = params["enc_node"], params["enc_edge"]
    dn_, de_ = params["dec_node"], params["dec_edge"]

    w16 = _w_spec((LAT, HID))
    w1h = _w_spec((1, HID))
    w1l = _w_spec((1, LAT))

    ns, nd, pn, f_s, f_d, f_n = _tc_call(
        _enc_node_body, N_NODES // BN,
        in_specs=[_row_spec(BN, LAT), _w_spec((LAT, HID)), w1h,
                  _w_spec((HID, LAT)), w1l, w1l, w1l,
                  w16, w16, w16, w16, w16, w16],
        out_shapes=[jax.ShapeDtypeStruct((N_NODES, HID), f32)] * 6,
        out_specs=[_row_spec(BN, HID)] * 6,
        name="tc_enc_node", interpret=interpret,
    )(vdata, en_["W1"], r2(en_["b1"]), en_["W2"], r2(en_["b2"]),
      r2(en_["g"]), r2(en_["bt"]),
      A_vs, A_vd, B_v, A_v0s, A_v0d, B_v0)

    eye8 = jnp.eye(8, dtype=f32)
    k8 = lambda w: jnp.kron(eye8, w)
    t8 = lambda b: jnp.tile(b, 8).reshape(1, -1)
    g3ce = pce["g"].reshape(1, 1, LAT)
    bt3ce = pce["bt"].reshape(1, 1, LAT)
    edp = jnp.reshape(edata, (EP8, 8 * HID))
    wp = _w_spec((BP, 8 * HID))
    wpl = _w_spec((BP, LAT))

    pe0, pec = _tc_call(
        _enc_edge_body, N_EDGES // BE,
        in_specs=[_row_spec(BP, 8 * HID), _w_spec((8 * HID, 8 * HID)),
                  _w_spec((1, 8 * HID)), _w_spec((8 * HID, 8 * LAT)),
                  _w_spec((1, 8 * LAT)), _w_spec((1, 1, LAT)),
                  _w_spec((1, 1, LAT)), _w_spec((8 * LAT, 8 * HID)),
                  _w_spec((8 * LAT, 8 * HID))],
        out_shapes=[jax.ShapeDtypeStruct((EP8, 8 * HID), f32)] * 2,
        out_specs=[_row_spec(BP, 8 * HID)] * 2,
        name="tc_enc_edge", interpret=interpret,
    )(edp, k8(ee_["W1"]), t8(ee_["b1"]), k8(ee_["W2"]), t8(ee_["b2"]),
      ee_["g"].reshape(1, 1, LAT), ee_["bt"].reshape(1, 1, LAT),
      k8(A_e0), k8(A_e))

    zeros_nh = jnp.zeros((N_NODES, HID), f32)
    zeros_nl = jnp.zeros((N_NODES, LAT), f32)

    if interpret:
        def do_gather(eb, ec, ns_, nd_):
            s_ = lax.dynamic_slice_in_dim(src, eb, ec)
            d_ = lax.dynamic_slice_in_dim(dst, eb, ec)
            return jnp.take(ns_, s_, axis=0) + jnp.take(nd_, d_, axis=0)

        def do_count():
            s = jax.ops.segment_sum(jnp.ones((N_EDGES, HID), f32), dst,
                                    num_segments=N_NODES)
            return s, jnp.zeros_like(s)

        def do_scatter(eb, ec, pb, x):
            d_ = lax.dynamic_slice_in_dim(dst, eb, ec)
            x_ = lax.dynamic_slice_in_dim(x, pb, ec) if x.shape[0] != ec else x
            s = jax.ops.segment_sum(x_, d_, num_segments=N_NODES)
            return s, jnp.zeros_like(s)
    else:
        def do_gather(eb, ec, ns_, nd_):
            out = _sc_gather_k(eb, ec)(ns_, nd_, src, dst)
            return out[0] if isinstance(out, (list, tuple)) else out

        def do_count():
            return _sc_count_k()(dst, zeros_nh)

        def do_scatter(eb, ec, pb, x):
            return _sc_scatter_k(eb, ec, pb)(x, dst, zeros_nl)

    cntp = do_count()

    def edge_specs(e_base, pec_is_half):
        off = e_base // BE
        full = pl.BlockSpec((BP, 8 * HID), lambda i, o=off: (i + o, 0))
        half = _row_spec(BP, 8 * HID)
        return [full, half if pec_is_half else full, half]

    ew = [t8(pce["b1"]), k8(pce["W2"]), t8(pce["b2"]), g3ce, bt3ce]
    ew_specs = [_w_spec((1, 8 * HID)), _w_spec((8 * HID, 8 * LAT)),
                _w_spec((1, 8 * LAT)), _w_spec((1, 1, LAT)),
                _w_spec((1, 1, LAT))]

    Aeb = k8(A_e)
    pec_halves = None
    for t in range(CORE_STEPS - 1):
        hsums = [jnp.reshape(do_gather(eb, ec, ns, nd), (ec // 8, 8 * HID))
                 for (eb, ec) in CHUNKS]
        ens, aggs, pec_new = [], [], []
        for ci, (eb, ec) in enumerate(CHUNKS):
            pec_in = pec if pec_halves is None else pec_halves[ci]
            en, pc = _tc_call(
                _edge_step_body, ec // BE,
                in_specs=edge_specs(eb, pec_halves is not None) + ew_specs
                + [_w_spec((8 * LAT, 8 * HID))],
                out_shapes=[jax.ShapeDtypeStruct((ec, LAT), f32),
                            jax.ShapeDtypeStruct((ec // 8, 8 * HID), f32)],
                out_specs=[_row_spec(BE, LAT), _row_spec(BP, 8 * HID)],
                name=f"tc_edge_step{t}_{ci}", interpret=interpret,
            )(pe0, pec_in, hsums[ci], *ew, Aeb)
            ens.append(en)
            pec_new.append(pc)
            aggs.append(do_scatter(eb, ec, 0, en))
        pec_halves = pec_new

        ns, nd, pn = _tc_call(
            _node_step_body, N_NODES // BN,
            in_specs=[_row_spec(BN, HID), _row_spec(BN, LAT),
                      _row_spec(BN, LAT), _row_spec(BN, LAT),
                      _row_spec(BN, LAT), _row_spec(BN, HID),
                      _row_spec(BN, HID), _row_spec(BN, HID),
                      _row_spec(BN, HID), _row_spec(BN, HID),
                      w1h, _w_spec((HID, LAT)), w1l, w1l, w1l,
                      w16, w16, w16, w16],
            out_shapes=[jax.ShapeDtypeStruct((N_NODES, HID), f32)] * 3,
            out_specs=[_row_spec(BN, HID)] * 3,
            name=f"tc_node_step{t}", interpret=interpret,
        )(pn, aggs[0][0], aggs[0][1], aggs[1][0], aggs[1][1],
          cntp[0], cntp[1], f_s, f_d, f_n,
          r2(pcn["b1"]), pcn["W2"], r2(pcn["b2"]), r2(pcn["g"]),
          r2(pcn["bt"]), B_agg, A_vs, A_vd, B_v)

    hs3 = [jnp.reshape(do_gather(eb, ec, ns, nd), (ec // 8, 8 * HID))
           for (eb, ec) in CHUNKS]
    pec3 = jnp.concatenate(pec_halves, axis=0)
    nb0 = CHUNKS[0][1] // BE
    en3 = _tc_call(
        _edge_core_body, N_EDGES // BE,
        in_specs=[_row_spec(BP, 8 * HID), _row_spec(BP, 8 * HID),
                  pl.BlockSpec((BP, 8 * HID),
                               lambda i: (jnp.minimum(i, nb0 - 1), 0)),
                  pl.BlockSpec((BP, 8 * HID),
                               lambda i: (jnp.maximum(i - nb0, 0), 0))]
        + ew_specs,
        out_shapes=jax.ShapeDtypeStruct((N_EDGES, LAT), f32),
        out_specs=_row_spec(BE, LAT),
        name="tc_edge_core_last", interpret=interpret,
    )(pe0, pec3, hs3[0], hs3[1], *ew)

    aggs3 = [do_scatter(eb, ec, eb, en3) for (eb, ec) in CHUNKS]

    e_out = _tc_call(
        _edge_dec_body, N_EDGES // BE,
        in_specs=[_row_spec(BE, LAT),
                  _w_spec((LAT, HID)), w1h, _w_spec((HID, LAT)),
                  w1l, w1l, w1l, _w_spec((LAT, LAT)), w1l],
        out_shapes=jax.ShapeDtypeStruct((N_EDGES, LAT), f32),
        out_specs=_row_spec(BE, LAT),
        name="tc_edge_dec", interpret=interpret,
    )(en3, de_["W1"], r2(de_["b1"]), de_["W2"], r2(de_["b2"]),
      r2(de_["g"]), r2(de_["bt"]),
      params["dec_edge_out_W"], r2(params["dec_edge_out_b"]))

    v_out = _tc_call(
        _node_last_body, N_NODES // BN,
        in_specs=[_row_spec(BN, HID), _row_spec(BN, LAT),
                  _row_spec(BN, LAT), _row_spec(BN, LAT),
                  _row_spec(BN, LAT), _row_spec(BN, HID),
                  _row_spec(BN, HID),
                  w1h, _w_spec((HID, LAT)), w1l, w1l, w1l, w16,
                  _w_spec((LAT, HID)), w1h, _w_spec((HID, LAT)),
                  w1l, w1l, w1l, _w_spec((LAT, LAT)), w1l],
        out_shapes=jax.ShapeDtypeStruct((N_NODES, LAT), f32),
        out_specs=_row_spec(BN, LAT),
        name="tc_node_last", interpret=interpret,
    )(pn, aggs3[0][0], aggs3[0][1], aggs3[1][0], aggs3[1][1],
      cntp[0], cntp[1],
      r2(pcn["b1"]), pcn["W2"], r2(pcn["b2"]), r2(pcn["g"]),
      r2(pcn["bt"]), B_agg,
      dn_["W1"], r2(dn_["b1"]), dn_["W2"], r2(dn_["b2"]),
      r2(dn_["g"]), r2(dn_["bt"]),
      params["dec_node_out_W"], r2(params["dec_node_out_b"]))

    return (v_out, e_out)


def kernel(vdata, edata, connectivity, cdata, metadata, params):
    return _forward_impl(vdata, edata, connectivity, params)


# speedup vs baseline: 1.7565x; 1.0484x over previous
import functools
import jax
import jax.numpy as jnp
from jax import lax
from jax.experimental import pallas as pl
from jax.experimental.pallas import tpu as pltpu
from jax.experimental.pallas import tpu_sc as plsc

N_NODES = 10000
N_EDGES = 160000
LAT = 128
HID = 16
CORE_STEPS = 3

BN = 1000
BE = 3200
BP = BE // 8
EP8 = N_EDGES // 8
NW = 32
EPW = N_EDGES // NW
NPT = N_NODES // 16
CH = 200
NCH = EPW // CH

_EPS = 1e-5


def _ln(h, g, bt):
    mu = jnp.mean(h, axis=-1, keepdims=True)
    var = jnp.mean((h - mu) * (h - mu), axis=-1, keepdims=True)
    return (h - mu) * lax.rsqrt(var + _EPS) * g + bt


def _dot(a, b):
    return jnp.dot(a, b, preferred_element_type=jnp.float32)


def _mlp_tail(h1pre, W2, b2, g, bt):
    h = jnp.maximum(h1pre, 0.0)
    h = jnp.maximum(_dot(h, W2) + b2, 0.0)
    return _ln(h, g, bt)


def _row_spec(b, d):
    return pl.BlockSpec((b, d), lambda i: (i, 0))


def _w_spec(shape):
    return pl.BlockSpec(shape, lambda i: tuple(0 for _ in shape))


def _enc_node_body(x, W1, b1, W2, b2, g, bt, Avs, Avd, Bv, Av0s, Av0d, Bv0,
                   ns_o, nd_o, pn_o, fs_o, fd_o, fn_o):
    h1 = _dot(x[...], W1[...]) + b1[...]
    v0 = _mlp_tail(h1, W2[...], b2[...], g[...], bt[...])
    fs = _dot(v0, Av0s[...])
    fd = _dot(v0, Av0d[...])
    fn = _dot(v0, Bv0[...])
    fs_o[...] = fs
    fd_o[...] = fd
    fn_o[...] = fn
    ns_o[...] = fs + _dot(v0, Avs[...])
    nd_o[...] = fd + _dot(v0, Avd[...])
    pn_o[...] = fn + _dot(v0, Bv[...])


def _enc_edge_body(xp, W1b, b1t, W2b, b2t, g3, bt3, A0b, A1b, pe0_o, pec_o):
    h1p = jnp.maximum(_dot(xp[...], W1b[...]) + b1t[...], 0.0)
    enw = jnp.maximum(_dot(h1p, W2b[...]) + b2t[...], 0.0)
    e3 = jnp.reshape(enw, (BP, 8, LAT))
    mu = jnp.mean(e3, axis=-1, keepdims=True)
    var = jnp.mean((e3 - mu) * (e3 - mu), axis=-1, keepdims=True)
    e3 = (e3 - mu) * lax.rsqrt(var + _EPS) * g3[...] + bt3[...]
    ef = jnp.reshape(e3, (BP, 8 * LAT))
    pe0_o[...] = _dot(ef, A0b[...])
    pec_o[...] = _dot(ef, A1b[...])


def _edge_step_body(pe0, pec, hs, b1t, W2b, b2t, g3, bt3, Aeb,
                    en_o, pec_o):
    h1p = jnp.maximum(pe0[...] + pec[...] + hs[...] + b1t[...], 0.0)
    enw = jnp.maximum(_dot(h1p, W2b[...]) + b2t[...], 0.0)
    e3 = jnp.reshape(enw, (BP, 8, LAT))
    mu = jnp.mean(e3, axis=-1, keepdims=True)
    var = jnp.mean((e3 - mu) * (e3 - mu), axis=-1, keepdims=True)
    e3 = (e3 - mu) * lax.rsqrt(var + _EPS) * g3[...] + bt3[...]
    en_o[...] = jnp.reshape(e3, (BE, LAT))
    pec_o[...] = _dot(jnp.reshape(e3, (BP, 8 * LAT)), Aeb[...])


def _edge_core_body(pe0, pec, hs0, hs1, b1t, W2b, b2t, g3, bt3, en_o):
    i = pl.program_id(0)
    nb0 = CHUNKS[0][1] // BE
    hs = jnp.where(i < nb0, hs0[...], hs1[...])
    h1p = jnp.maximum(pe0[...] + pec[...] + hs + b1t[...], 0.0)
    enw = jnp.maximum(_dot(h1p, W2b[...]) + b2t[...], 0.0)
    e3 = jnp.reshape(enw, (BP, 8, LAT))
    mu = jnp.mean(e3, axis=-1, keepdims=True)
    var = jnp.mean((e3 - mu) * (e3 - mu), axis=-1, keepdims=True)
    e3 = (e3 - mu) * lax.rsqrt(var + _EPS) * g3[...] + bt3[...]
    en_o[...] = jnp.reshape(e3, (BE, LAT))


def _edge_dec_body(en, dW1, db1, dW2, db2, dg, dbt, oW, ob, eout_o):
    d1 = _dot(en[...], dW1[...]) + db1[...]
    dec = _mlp_tail(d1, dW2[...], db2[...], dg[...], dbt[...])
    eout_o[...] = _dot(dec, oW[...]) + ob[...]


def _node_step_body(pn, a0, a1, a2, a3, c0, c1, fs, fd, fn, b1, W2, b2, g, bt,
                    Bagg, Avs, Avd, Bv, ns_o, nd_o, pn_o):
    cm = jnp.maximum(c0[...] + c1[...], 1.0)[:, 0:1]
    agg = (a0[...] + a1[...] + a2[...] + a3[...]) / cm
    h1 = pn[...] + _dot(agg, Bagg[...]) + b1[...]
    vn = _mlp_tail(h1, W2[...], b2[...], g[...], bt[...])
    ns_o[...] = fs[...] + _dot(vn, Avs[...])
    nd_o[...] = fd[...] + _dot(vn, Avd[...])
    pn_o[...] = fn[...] + _dot(vn, Bv[...])


def _node_last_body(pn, a0, a1, a2, a3, c0, c1, b1, W2, b2, g, bt, Bagg,
                    dW1, db1, dW2, db2, dg, dbt, oW, ob, vout_o):
    cm = jnp.maximum(c0[...] + c1[...], 1.0)[:, 0:1]
    agg = (a0[...] + a1[...] + a2[...] + a3[...]) / cm
    h1 = pn[...] + _dot(agg, Bagg[...]) + b1[...]
    vn = _mlp_tail(h1, W2[...], b2[...], g[...], bt[...])
    d1 = _dot(vn, dW1[...]) + db1[...]
    dec = _mlp_tail(d1, dW2[...], db2[...], dg[...], dbt[...])
    vout_o[...] = _dot(dec, oW[...]) + ob[...]


def _tc_call(body, grid, in_specs, out_shapes, out_specs, name,
             interpret=False):
    return pl.pallas_call(
        body,
        grid=(grid,),
        in_specs=in_specs,
        out_specs=out_specs,
        out_shape=out_shapes,
        interpret=interpret,
        name=name,
    )


_MESHF = plsc.VectorSubcoreMesh
_CPF = pltpu.CompilerParams


@functools.cache
def _sc_gather_k(e_base, e_cnt):
    epw = e_cnt // NW
    mesh = _MESHF(core_axis_name="c", subcore_axis_name="s")

    @functools.partial(
        pl.kernel,
        out_type=[jax.ShapeDtypeStruct((e_cnt, HID), jnp.float32)],
        mesh=mesh,
        compiler_params=_CPF(use_tc_tiling_on_sc=False),
        scratch_types=[pltpu.VMEM((epw,), jnp.int32),
                       pltpu.VMEM((epw,), jnp.int32),
                       pltpu.VMEM((epw, HID), jnp.float32),
                       pltpu.VMEM((epw, HID), jnp.float32),
                       pltpu.SemaphoreType.DMA],
        name=f"sc_gather_{e_base}_{e_cnt}",
    )
    def sc_gather(ns_h, nd_h, src_h, dst_h, hs_h, idx_s, idx_d, rs_v, rd_v,
                  sem):
        wid = lax.axis_index("s") * 2 + lax.axis_index("c")
        lsl = pl.ds(wid * epw, epw)
        gsl = pl.ds(e_base + wid * epw, epw)
        pltpu.sync_copy(src_h.at[gsl], idx_s)
        pltpu.sync_copy(dst_h.at[gsl], idx_d)
        cps = pltpu.async_copy(ns_h.at[idx_s], rs_v, sem)
        cpd = pltpu.async_copy(nd_h.at[idx_d], rd_v, sem)
        cps.wait()
        cpd.wait()

        def add4(i, _):
            for k in range(4):
                r = i * 4 + k
                rs_v[r, :] = rs_v[r, :] + rd_v[r, :]
            return 0

        lax.fori_loop(0, epw // 4, add4, 0, unroll=False)
        pltpu.sync_copy(rs_v, hs_h.at[lsl])

    return sc_gather


@functools.cache
def _sc_count_k():
    mesh = _MESHF(core_axis_name="c", subcore_axis_name="s")

    @functools.partial(
        pl.kernel,
        out_type=[jax.ShapeDtypeStruct((N_NODES, HID), jnp.float32),
                  jax.ShapeDtypeStruct((N_NODES, HID), jnp.float32)],
        mesh=mesh,
        compiler_params=_CPF(use_tc_tiling_on_sc=False),
        scratch_types=[pltpu.VMEM((EPW,), jnp.int32),
                       pltpu.VMEM((EPW, HID), jnp.float32),
                       pltpu.VMEM_SHARED((N_NODES, HID), jnp.float32)],
        name="sc_count",
    )
    def sc_count(dst_h, zeros_h, out0_h, out1_h, idx_v, pa_v, acc):
        sid = lax.axis_index("s")
        cid = lax.axis_index("c")
        base = (sid * 2 + cid) * EPW
        nsl = pl.ds(sid * NPT, NPT)
        pltpu.sync_copy(zeros_h.at[nsl], acc.at[nsl])

        one = jnp.ones((HID,), jnp.float32)

        def fill4(i, _):
            for k in range(4):
                pa_v[i * 4 + k, :] = one
            return 0

        lax.fori_loop(0, EPW // 4, fill4, 0, unroll=False)
        plsc.subcore_barrier()
        pltpu.sync_copy(dst_h.at[pl.ds(base, EPW)], idx_v)
        pltpu.sync_copy(pa_v, acc.at[idx_v], add=True)
        plsc.subcore_barrier()

        @pl.when(cid == 0)
        def _():
            pltpu.sync_copy(acc.at[nsl], out0_h.at[nsl])

        @pl.when(cid == 1)
        def _():
            pltpu.sync_copy(acc.at[nsl], out1_h.at[nsl])

    return sc_count


@functools.cache
def _sc_scatter_k(e_base, e_cnt, pa_base):
    epw = e_cnt // NW
    ch = 120 if epw % 120 == 0 else 80
    nch = epw // ch
    mesh = _MESHF(core_axis_name="c", subcore_axis_name="s")

    @functools.partial(
        pl.kernel,
        out_type=[jax.ShapeDtypeStruct((N_NODES, LAT), jnp.float32),
                  jax.ShapeDtypeStruct((N_NODES, LAT), jnp.float32)],
        mesh=mesh,
        compiler_params=_CPF(use_tc_tiling_on_sc=False),
        scratch_types=[pltpu.VMEM((nch, ch), jnp.int32),
                       pltpu.VMEM((2, ch, LAT), jnp.float32),
                       pltpu.VMEM_SHARED((N_NODES, LAT), jnp.float32),
                       pltpu.SemaphoreType.DMA,
                       pltpu.SemaphoreType.DMA,
                       pltpu.SemaphoreType.DMA,
                       pltpu.SemaphoreType.DMA,
                       pltpu.SemaphoreType.DMA],
        name=f"sc_scatter_{e_base}_{e_cnt}",
    )
    def sc_scatter(pa_h, dst_h, zeros_h, out0_h, out1_h, idx2, pav, acc,
                   semi, seml0, seml1, sems0, sems1):
        sid = lax.axis_index("s")
        cid = lax.axis_index("c")
        lbase = (sid * 2 + cid) * epw
        gbase = e_base + lbase
        pbase = pa_base + lbase
        nsl = pl.ds(sid * NPT, NPT)
        icps = [pltpu.async_copy(dst_h.at[pl.ds(gbase + j * ch, ch)],
                                 idx2.at[j], semi) for j in range(nch)]
        pltpu.sync_copy(zeros_h.at[nsl], acc.at[nsl])
        for c in icps:
            c.wait()
        plsc.subcore_barrier()
        seml = [seml0, seml1]
        lds = [None] * nch
        lds[0] = pltpu.async_copy(pa_h.at[pl.ds(pbase, ch)], pav.at[0],
                                  seml[0])
        for j in range(nch):
            lds[j].wait()
            if j + 1 < nch:
                lds[j + 1] = pltpu.async_copy(
                    pa_h.at[pl.ds(pbase + (j + 1) * ch, ch)],
                    pav.at[(j + 1) % 2], seml[(j + 1) % 2])
            pltpu.sync_copy(pav.at[j % 2], acc.at[idx2.at[j]], add=True)
        plsc.subcore_barrier()

        @pl.when(cid == 0)
        def _():
            pltpu.sync_copy(acc.at[nsl], out0_h.at[nsl])

        @pl.when(cid == 1)
        def _():
            pltpu.sync_copy(acc.at[nsl], out1_h.at[nsl])

    return sc_scatter


CHUNKS = ((0, 96000), (96000, 64000))


def _forward_impl(vdata, edata, connectivity, params, interpret=False):
    f32 = jnp.float32
    src = connectivity[0]
    dst = connectivity[1]

    pce, pcn = params["core_edge"], params["core_node"]
    W1ce, W1cn = pce["W1"], pcn["W1"]
    A_e0, A_e = W1ce[0:LAT], W1ce[LAT:2 * LAT]
    A_v0s, A_vs = W1ce[2 * LAT:3 * LAT], W1ce[3 * LAT:4 * LAT]
    A_v0d, A_vd = W1ce[4 * LAT:5 * LAT], W1ce[5 * LAT:6 * LAT]
    B_v0, B_v, B_agg = W1cn[0:LAT], W1cn[LAT:2 * LAT], W1cn[2 * LAT:3 * LAT]

    def r2(x):
        return x.reshape(1, -1)

    en_, ee_ = params["enc_node"], params["enc_edge"]
    dn_, de_ = params["dec_node"], params["dec_edge"]

    w16 = _w_spec((LAT, HID))
    w1h = _w_spec((1, HID))
    w1l = _w_spec((1, LAT))

    ns, nd, pn, f_s, f_d, f_n = _tc_call(
        _enc_node_body, N_NODES // BN,
        in_specs=[_row_spec(BN, LAT), _w_spec((LAT, HID)), w1h,
                  _w_spec((HID, LAT)), w1l, w1l, w1l,
                  w16, w16, w16, w16, w16, w16],
        out_shapes=[jax.ShapeDtypeStruct((N_NODES, HID), f32)] * 6,
        out_specs=[_row_spec(BN, HID)] * 6,
        name="tc_enc_node", interpret=interpret,
    )(vdata, en_["W1"], r2(en_["b1"]), en_["W2"], r2(en_["b2"]),
      r2(en_["g"]), r2(en_["bt"]),
      A_vs, A_vd, B_v, A_v0s, A_v0d, B_v0)

    eye8 = jnp.eye(8, dtype=f32)
    k8 = lambda w: jnp.kron(eye8, w)
    t8 = lambda b: jnp.tile(b, 8).reshape(1, -1)
    g3ce = pce["g"].reshape(1, 1, LAT)
    bt3ce = pce["bt"].reshape(1, 1, LAT)
    edp = jnp.reshape(edata, (EP8, 8 * HID))
    wp = _w_spec((BP, 8 * HID))
    wpl = _w_spec((BP, LAT))

    pe0, pec = _tc_call(
        _enc_edge_body, N_EDGES // BE,
        in_specs=[_row_spec(BP, 8 * HID), _w_spec((8 * HID, 8 * HID)),
                  _w_spec((1, 8 * HID)), _w_spec((8 * HID, 8 * LAT)),
                  _w_spec((1, 8 * LAT)), _w_spec((1, 1, LAT)),
                  _w_spec((1, 1, LAT)), _w_spec((8 * LAT, 8 * HID)),
                  _w_spec((8 * LAT, 8 * HID))],
        out_shapes=[jax.ShapeDtypeStruct((EP8, 8 * HID), f32)] * 2,
        out_specs=[_row_spec(BP, 8 * HID)] * 2,
        name="tc_enc_edge", interpret=interpret,
    )(edp, k8(ee_["W1"]), t8(ee_["b1"]), k8(ee_["W2"]), t8(ee_["b2"]),
      ee_["g"].reshape(1, 1, LAT), ee_["bt"].reshape(1, 1, LAT),
      k8(A_e0), k8(A_e))

    zeros_nh = jnp.zeros((N_NODES, HID), f32)
    zeros_nl = jnp.zeros((N_NODES, LAT), f32)

    if interpret:
        def do_gather(eb, ec, ns_, nd_):
            s_ = lax.dynamic_slice_in_dim(src, eb, ec)
            d_ = lax.dynamic_slice_in_dim(dst, eb, ec)
            return jnp.take(ns_, s_, axis=0) + jnp.take(nd_, d_, axis=0)

        def do_count():
            s = jax.ops.segment_sum(jnp.ones((N_EDGES, HID), f32), dst,
                                    num_segments=N_NODES)
            return s, jnp.zeros_like(s)

        def do_scatter(eb, ec, pb, x):
            d_ = lax.dynamic_slice_in_dim(dst, eb, ec)
            x_ = lax.dynamic_slice_in_dim(x, pb, ec) if x.shape[0] != ec else x
            s = jax.ops.segment_sum(x_, d_, num_segments=N_NODES)
            return s, jnp.zeros_like(s)
    else:
        def do_gather(eb, ec, ns_, nd_):
            out = _sc_gather_k(eb, ec)(ns_, nd_, src, dst)
            return out[0] if isinstance(out, (list, tuple)) else out

        def do_count():
            return _sc_count_k()(dst, zeros_nh)

        def do_scatter(eb, ec, pb, x):
            return _sc_scatter_k(eb, ec, pb)(x, dst, zeros_nl)

    cntp = do_count()

    def edge_specs(e_base, pec_is_half):
        off = e_base // BE
        full = pl.BlockSpec((BP, 8 * HID), lambda i, o=off: (i + o, 0))
        half = _row_spec(BP, 8 * HID)
        return [full, half if pec_is_half else full, half]

    ew = [t8(pce["b1"]), k8(pce["W2"]), t8(pce["b2"]), g3ce, bt3ce]
    ew_specs = [_w_spec((1, 8 * HID)), _w_spec((8 * HID, 8 * LAT)),
                _w_spec((1, 8 * LAT)), _w_spec((1, 1, LAT)),
                _w_spec((1, 1, LAT))]

    Aeb = k8(A_e)
    pec_halves = None
    for t in range(CORE_STEPS - 1):
        hsums = [jnp.reshape(do_gather(eb, ec, ns, nd), (ec // 8, 8 * HID))
                 for (eb, ec) in CHUNKS]
        ens, aggs, pec_new = [], [], []
        for ci, (eb, ec) in enumerate(CHUNKS):
            pec_in = pec if pec_halves is None else pec_halves[ci]
            en, pc = _tc_call(
                _edge_step_body, ec // BE,
                in_specs=edge_specs(eb, pec_halves is not None) + ew_specs
                + [_w_spec((8 * LAT, 8 * HID))],
                out_shapes=[jax.ShapeDtypeStruct((ec, LAT), f32),
                            jax.ShapeDtypeStruct((ec // 8, 8 * HID), f32)],
                out_specs=[_row_spec(BE, LAT), _row_spec(BP, 8 * HID)],
                name=f"tc_edge_step{t}_{ci}", interpret=interpret,
            )(pe0, pec_in, hsums[ci], *ew, Aeb)
            ens.append(en)
            pec_new.append(pc)
            aggs.append(do_scatter(eb, ec, 0, en))
        pec_halves = pec_new

        ns, nd, pn = _tc_call(
            _node_step_body, N_NODES // BN,
            in_specs=[_row_spec(BN, HID), _row_spec(BN, LAT),
                      _row_spec(BN, LAT), _row_spec(BN, LAT),
                      _row_spec(BN, LAT), _row_spec(BN, HID),
                      _row_spec(BN, HID), _row_spec(BN, HID),
                      _row_spec(BN, HID), _row_spec(BN, HID),
                      w1h, _w_spec((HID, LAT)), w1l, w1l, w1l,
                      w16, w16, w16, w16],
            out_shapes=[jax.ShapeDtypeStruct((N_NODES, HID), f32)] * 3,
            out_specs=[_row_spec(BN, HID)] * 3,
            name=f"tc_node_step{t}", interpret=interpret,
        )(pn, aggs[0][0], aggs[0][1], aggs[1][0], aggs[1][1],
          cntp[0], cntp[1], f_s, f_d, f_n,
          r2(pcn["b1"]), pcn["W2"], r2(pcn["b2"]), r2(pcn["g"]),
          r2(pcn["bt"]), B_agg, A_vs, A_vd, B_v)

    hs3 = [jnp.reshape(do_gather(eb, ec, ns, nd), (ec // 8, 8 * HID))
           for (eb, ec) in CHUNKS]
    pec3 = jnp.concatenate(pec_halves, axis=0)
    nb0 = CHUNKS[0][1] // BE
    en3 = _tc_call(
        _edge_core_body, N_EDGES // BE,
        in_specs=[_row_spec(BP, 8 * HID), _row_spec(BP, 8 * HID),
                  pl.BlockSpec((BP, 8 * HID),
                               lambda i: (jnp.minimum(i, nb0 - 1), 0)),
                  pl.BlockSpec((BP, 8 * HID),
                               lambda i: (jnp.maximum(i - nb0, 0), 0))]
        + ew_specs,
        out_shapes=jax.ShapeDtypeStruct((N_EDGES, LAT), f32),
        out_specs=_row_spec(BE, LAT),
        name="tc_edge_core_last", interpret=interpret,
    )(pe0, pec3, hs3[0], hs3[1], *ew)

    aggs3 = [do_scatter(eb, ec, eb, en3) for (eb, ec) in CHUNKS]

    e_out = _tc_call(
        _edge_dec_body, N_EDGES // BE,
        in_specs=[_row_spec(BE, LAT),
                  _w_spec((LAT, HID)), w1h, _w_spec((HID, LAT)),
                  w1l, w1l, w1l, _w_spec((LAT, LAT)), w1l],
        out_shapes=jax.ShapeDtypeStruct((N_EDGES, LAT), f32),
        out_specs=_row_spec(BE, LAT),
        name="tc_edge_dec", interpret=interpret,
    )(en3, de_["W1"], r2(de_["b1"]), de_["W2"], r2(de_["b2"]),
      r2(de_["g"]), r2(de_["bt"]),
      params["dec_edge_out_W"], r2(params["dec_edge_out_b"]))

    v_out = _tc_call(
        _node_last_body, N_NODES // BN,
        in_specs=[_row_spec(BN, HID), _row_spec(BN, LAT),
                  _row_spec(BN, LAT), _row_spec(BN, LAT),
                  _row_spec(BN, LAT), _row_spec(BN, HID),
                  _row_spec(BN, HID),
                  w1h, _w_spec((HID, LAT)), w1l, w1l, w1l, w16,
                  _w_spec((LAT, HID)), w1h, _w_spec((HID, LAT)),
                  w1l, w1l, w1l, _w_spec((LAT, LAT)), w1l],
        out_shapes=jax.ShapeDtypeStruct((N_NODES, LAT), f32),
        out_specs=_row_spec(BN, LAT),
        name="tc_node_last", interpret=interpret,
    )(pn, aggs3[0][0], aggs3[0][1], aggs3[1][0], aggs3[1][1],
      cntp[0], cntp[1],
      r2(pcn["b1"]), pcn["W2"], r2(pcn["b2"]), r2(pcn["g"]),
      r2(pcn["bt"]), B_agg,
      dn_["W1"], r2(dn_["b1"]), dn_["W2"], r2(dn_["b2"]),
      r2(dn_["g"]), r2(dn_["bt"]),
      params["dec_node_out_W"], r2(params["dec_node_out_b"]))

    return (v_out, e_out)


def kernel(vdata, edata, connectivity, cdata, metadata, params):
    return _forward_impl(vdata, edata, connectivity, params)


# speedup vs baseline: 1.8207x; 1.0365x over previous
import functools
import jax
import jax.numpy as jnp
from jax import lax
from jax.experimental import pallas as pl
from jax.experimental.pallas import tpu as pltpu
from jax.experimental.pallas import tpu_sc as plsc

N_NODES = 10000
N_EDGES = 160000
LAT = 128
HID = 16
CORE_STEPS = 3

BN = 1000
BE = 6400
BP = BE // 8
EP8 = N_EDGES // 8
NW = 32
EPW = N_EDGES // NW
NPT = N_NODES // 16
CH = 200
NCH = EPW // CH

_EPS = 1e-5


def _ln(h, g, bt):
    mu = jnp.mean(h, axis=-1, keepdims=True)
    var = jnp.mean((h - mu) * (h - mu), axis=-1, keepdims=True)
    return (h - mu) * lax.rsqrt(var + _EPS) * g + bt


def _dot(a, b):
    return jnp.dot(a, b, preferred_element_type=jnp.float32)


def _mlp_tail(h1pre, W2, b2, g, bt):
    h = jnp.maximum(h1pre, 0.0)
    h = jnp.maximum(_dot(h, W2) + b2, 0.0)
    return _ln(h, g, bt)


def _row_spec(b, d):
    return pl.BlockSpec((b, d), lambda i: (i, 0))


def _w_spec(shape):
    return pl.BlockSpec(shape, lambda i: tuple(0 for _ in shape))


def _enc_node_body(x, W1, b1, W2, b2, g, bt, Avs, Avd, Bv, Av0s, Av0d, Bv0,
                   ns_o, nd_o, pn_o, fs_o, fd_o, fn_o):
    h1 = _dot(x[...], W1[...]) + b1[...]
    v0 = _mlp_tail(h1, W2[...], b2[...], g[...], bt[...])
    fs = _dot(v0, Av0s[...])
    fd = _dot(v0, Av0d[...])
    fn = _dot(v0, Bv0[...])
    fs_o[...] = fs
    fd_o[...] = fd
    fn_o[...] = fn
    ns_o[...] = fs + _dot(v0, Avs[...])
    nd_o[...] = fd + _dot(v0, Avd[...])
    pn_o[...] = fn + _dot(v0, Bv[...])


def _enc_edge_body(xp, W1b, b1t, W2b, b2t, g3, bt3, A0b, A1b, pe0_o, pec_o):
    h1p = jnp.maximum(_dot(xp[...], W1b[...]) + b1t[...], 0.0)
    enw = jnp.maximum(_dot(h1p, W2b[...]) + b2t[...], 0.0)
    e3 = jnp.reshape(enw, (BP, 8, LAT))
    mu = jnp.mean(e3, axis=-1, keepdims=True)
    var = jnp.mean((e3 - mu) * (e3 - mu), axis=-1, keepdims=True)
    e3 = (e3 - mu) * lax.rsqrt(var + _EPS) * g3[...] + bt3[...]
    ef = jnp.reshape(e3, (BP, 8 * LAT))
    pe0_o[...] = _dot(ef, A0b[...])
    pec_o[...] = _dot(ef, A1b[...])


def _edge_step_body(pe0, pec, hs, b1t, W2b, b2t, g3, bt3, Aeb,
                    en_o, pec_o):
    h1p = jnp.maximum(pe0[...] + pec[...] + hs[...] + b1t[...], 0.0)
    enw = jnp.maximum(_dot(h1p, W2b[...]) + b2t[...], 0.0)
    e3 = jnp.reshape(enw, (BP, 8, LAT))
    mu = jnp.mean(e3, axis=-1, keepdims=True)
    var = jnp.mean((e3 - mu) * (e3 - mu), axis=-1, keepdims=True)
    e3 = (e3 - mu) * lax.rsqrt(var + _EPS) * g3[...] + bt3[...]
    en_o[...] = jnp.reshape(e3, (BE, LAT))
    pec_o[...] = _dot(jnp.reshape(e3, (BP, 8 * LAT)), Aeb[...])


def _edge_core_body(pe0, pec, hs0, hs1, b1t, W2b, b2t, g3, bt3, en_o):
    i = pl.program_id(0)
    nb0 = CHUNKS[0][1] // BE
    hs = jnp.where(i < nb0, hs0[...], hs1[...])
    h1p = jnp.maximum(pe0[...] + pec[...] + hs + b1t[...], 0.0)
    enw = jnp.maximum(_dot(h1p, W2b[...]) + b2t[...], 0.0)
    e3 = jnp.reshape(enw, (BP, 8, LAT))
    mu = jnp.mean(e3, axis=-1, keepdims=True)
    var = jnp.mean((e3 - mu) * (e3 - mu), axis=-1, keepdims=True)
    e3 = (e3 - mu) * lax.rsqrt(var + _EPS) * g3[...] + bt3[...]
    en_o[...] = jnp.reshape(e3, (BE, LAT))


def _edge_dec_body(en, dW1, db1, dW2, db2, dg, dbt, oW, ob, eout_o):
    d1 = _dot(en[...], dW1[...]) + db1[...]
    dec = _mlp_tail(d1, dW2[...], db2[...], dg[...], dbt[...])
    eout_o[...] = _dot(dec, oW[...]) + ob[...]


def _node_step_body(pn, a0, a1, a2, a3, c0, c1, fs, fd, fn, b1, W2, b2, g, bt,
                    Bagg, Avs, Avd, Bv, ns_o, nd_o, pn_o):
    cm = jnp.maximum(c0[...] + c1[...], 1.0)[:, 0:1]
    agg = (a0[...] + a1[...] + a2[...] + a3[...]) / cm
    h1 = pn[...] + _dot(agg, Bagg[...]) + b1[...]
    vn = _mlp_tail(h1, W2[...], b2[...], g[...], bt[...])
    ns_o[...] = fs[...] + _dot(vn, Avs[...])
    nd_o[...] = fd[...] + _dot(vn, Avd[...])
    pn_o[...] = fn[...] + _dot(vn, Bv[...])


def _node_last_body(pn, a0, a1, a2, a3, c0, c1, b1, W2, b2, g, bt, Bagg,
                    dW1, db1, dW2, db2, dg, dbt, oW, ob, vout_o):
    cm = jnp.maximum(c0[...] + c1[...], 1.0)[:, 0:1]
    agg = (a0[...] + a1[...] + a2[...] + a3[...]) / cm
    h1 = pn[...] + _dot(agg, Bagg[...]) + b1[...]
    vn = _mlp_tail(h1, W2[...], b2[...], g[...], bt[...])
    d1 = _dot(vn, dW1[...]) + db1[...]
    dec = _mlp_tail(d1, dW2[...], db2[...], dg[...], dbt[...])
    vout_o[...] = _dot(dec, oW[...]) + ob[...]


def _tc_call(body, grid, in_specs, out_shapes, out_specs, name,
             interpret=False):
    return pl.pallas_call(
        body,
        grid=(grid,),
        in_specs=in_specs,
        out_specs=out_specs,
        out_shape=out_shapes,
        interpret=interpret,
        name=name,
    )


_MESHF = plsc.VectorSubcoreMesh
_CPF = pltpu.CompilerParams


@functools.cache
def _sc_gather_k(e_base, e_cnt):
    epw = e_cnt // NW
    mesh = _MESHF(core_axis_name="c", subcore_axis_name="s")

    @functools.partial(
        pl.kernel,
        out_type=[jax.ShapeDtypeStruct((e_cnt, HID), jnp.float32)],
        mesh=mesh,
        compiler_params=_CPF(use_tc_tiling_on_sc=False),
        scratch_types=[pltpu.VMEM((epw,), jnp.int32),
                       pltpu.VMEM((epw,), jnp.int32),
                       pltpu.VMEM((epw, HID), jnp.float32),
                       pltpu.VMEM((epw, HID), jnp.float32),
                       pltpu.SemaphoreType.DMA],
        name=f"sc_gather_{e_base}_{e_cnt}",
    )
    def sc_gather(ns_h, nd_h, src_h, dst_h, hs_h, idx_s, idx_d, rs_v, rd_v,
                  sem):
        wid = lax.axis_index("s") * 2 + lax.axis_index("c")
        lsl = pl.ds(wid * epw, epw)
        gsl = pl.ds(e_base + wid * epw, epw)
        pltpu.sync_copy(src_h.at[gsl], idx_s)
        pltpu.sync_copy(dst_h.at[gsl], idx_d)
        cps = pltpu.async_copy(ns_h.at[idx_s], rs_v, sem)
        cpd = pltpu.async_copy(nd_h.at[idx_d], rd_v, sem)
        cps.wait()
        cpd.wait()

        def add4(i, _):
            for k in range(4):
                r = i * 4 + k
                rs_v[r, :] = rs_v[r, :] + rd_v[r, :]
            return 0

        lax.fori_loop(0, epw // 4, add4, 0, unroll=False)
        pltpu.sync_copy(rs_v, hs_h.at[lsl])

    return sc_gather


@functools.cache
def _sc_count_k():
    mesh = _MESHF(core_axis_name="c", subcore_axis_name="s")

    @functools.partial(
        pl.kernel,
        out_type=[jax.ShapeDtypeStruct((N_NODES, HID), jnp.float32),
                  jax.ShapeDtypeStruct((N_NODES, HID), jnp.float32)],
        mesh=mesh,
        compiler_params=_CPF(use_tc_tiling_on_sc=False),
        scratch_types=[pltpu.VMEM((EPW,), jnp.int32),
                       pltpu.VMEM((EPW, HID), jnp.float32),
                       pltpu.VMEM_SHARED((N_NODES, HID), jnp.float32)],
        name="sc_count",
    )
    def sc_count(dst_h, zeros_h, out0_h, out1_h, idx_v, pa_v, acc):
        sid = lax.axis_index("s")
        cid = lax.axis_index("c")
        base = (sid * 2 + cid) * EPW
        nsl = pl.ds(sid * NPT, NPT)
        pltpu.sync_copy(zeros_h.at[nsl], acc.at[nsl])

        one = jnp.ones((HID,), jnp.float32)

        def fill4(i, _):
            for k in range(4):
                pa_v[i * 4 + k, :] = one
            return 0

        lax.fori_loop(0, EPW // 4, fill4, 0, unroll=False)
        plsc.subcore_barrier()
        pltpu.sync_copy(dst_h.at[pl.ds(base, EPW)], idx_v)
        pltpu.sync_copy(pa_v, acc.at[idx_v], add=True)
        plsc.subcore_barrier()

        @pl.when(cid == 0)
        def _():
            pltpu.sync_copy(acc.at[nsl], out0_h.at[nsl])

        @pl.when(cid == 1)
        def _():
            pltpu.sync_copy(acc.at[nsl], out1_h.at[nsl])

    return sc_count


@functools.cache
def _sc_scatter_k(e_base, e_cnt, pa_base):
    epw = e_cnt // NW
    ch = 120 if epw % 120 == 0 else 80
    nch = epw // ch
    mesh = _MESHF(core_axis_name="c", subcore_axis_name="s")

    @functools.partial(
        pl.kernel,
        out_type=[jax.ShapeDtypeStruct((N_NODES, LAT), jnp.float32),
                  jax.ShapeDtypeStruct((N_NODES, LAT), jnp.float32)],
        mesh=mesh,
        compiler_params=_CPF(use_tc_tiling_on_sc=False),
        scratch_types=[pltpu.VMEM((nch, ch), jnp.int32),
                       pltpu.VMEM((2, ch, LAT), jnp.float32),
                       pltpu.VMEM_SHARED((N_NODES, LAT), jnp.float32),
                       pltpu.SemaphoreType.DMA,
                       pltpu.SemaphoreType.DMA,
                       pltpu.SemaphoreType.DMA,
                       pltpu.SemaphoreType.DMA,
                       pltpu.SemaphoreType.DMA],
        name=f"sc_scatter_{e_base}_{e_cnt}",
    )
    def sc_scatter(pa_h, dst_h, zeros_h, out0_h, out1_h, idx2, pav, acc,
                   semi, seml0, seml1, sems0, sems1):
        sid = lax.axis_index("s")
        cid = lax.axis_index("c")
        lbase = (sid * 2 + cid) * epw
        gbase = e_base + lbase
        pbase = pa_base + lbase
        nsl = pl.ds(sid * NPT, NPT)
        icps = [pltpu.async_copy(dst_h.at[pl.ds(gbase + j * ch, ch)],
                                 idx2.at[j], semi) for j in range(nch)]
        pltpu.sync_copy(zeros_h.at[nsl], acc.at[nsl])
        for c in icps:
            c.wait()
        plsc.subcore_barrier()
        seml = [seml0, seml1]
        lds = [None] * nch
        lds[0] = pltpu.async_copy(pa_h.at[pl.ds(pbase, ch)], pav.at[0],
                                  seml[0])
        for j in range(nch):
            lds[j].wait()
            if j + 1 < nch:
                lds[j + 1] = pltpu.async_copy(
                    pa_h.at[pl.ds(pbase + (j + 1) * ch, ch)],
                    pav.at[(j + 1) % 2], seml[(j + 1) % 2])
            pltpu.sync_copy(pav.at[j % 2], acc.at[idx2.at[j]], add=True)
        plsc.subcore_barrier()

        @pl.when(cid == 0)
        def _():
            pltpu.sync_copy(acc.at[nsl], out0_h.at[nsl])

        @pl.when(cid == 1)
        def _():
            pltpu.sync_copy(acc.at[nsl], out1_h.at[nsl])

    return sc_scatter


CHUNKS = ((0, 96000), (96000, 64000))


def _forward_impl(vdata, edata, connectivity, params, interpret=False):
    f32 = jnp.float32
    src = connectivity[0]
    dst = connectivity[1]

    pce, pcn = params["core_edge"], params["core_node"]
    W1ce, W1cn = pce["W1"], pcn["W1"]
    A_e0, A_e = W1ce[0:LAT], W1ce[LAT:2 * LAT]
    A_v0s, A_vs = W1ce[2 * LAT:3 * LAT], W1ce[3 * LAT:4 * LAT]
    A_v0d, A_vd = W1ce[4 * LAT:5 * LAT], W1ce[5 * LAT:6 * LAT]
    B_v0, B_v, B_agg = W1cn[0:LAT], W1cn[LAT:2 * LAT], W1cn[2 * LAT:3 * LAT]

    def r2(x):
        return x.reshape(1, -1)

    en_, ee_ = params["enc_node"], params["enc_edge"]
    dn_, de_ = params["dec_node"], params["dec_edge"]

    w16 = _w_spec((LAT, HID))
    w1h = _w_spec((1, HID))
    w1l = _w_spec((1, LAT))

    ns, nd, pn, f_s, f_d, f_n = _tc_call(
        _enc_node_body, N_NODES // BN,
        in_specs=[_row_spec(BN, LAT), _w_spec((LAT, HID)), w1h,
                  _w_spec((HID, LAT)), w1l, w1l, w1l,
                  w16, w16, w16, w16, w16, w16],
        out_shapes=[jax.ShapeDtypeStruct((N_NODES, HID), f32)] * 6,
        out_specs=[_row_spec(BN, HID)] * 6,
        name="tc_enc_node", interpret=interpret,
    )(vdata, en_["W1"], r2(en_["b1"]), en_["W2"], r2(en_["b2"]),
      r2(en_["g"]), r2(en_["bt"]),
      A_vs, A_vd, B_v, A_v0s, A_v0d, B_v0)

    eye8 = jnp.eye(8, dtype=f32)
    k8 = lambda w: jnp.kron(eye8, w)
    t8 = lambda b: jnp.tile(b, 8).reshape(1, -1)
    g3ce = pce["g"].reshape(1, 1, LAT)
    bt3ce = pce["bt"].reshape(1, 1, LAT)
    edp = jnp.reshape(edata, (EP8, 8 * HID))
    wp = _w_spec((BP, 8 * HID))
    wpl = _w_spec((BP, LAT))

    pe0, pec = _tc_call(
        _enc_edge_body, N_EDGES // BE,
        in_specs=[_row_spec(BP, 8 * HID), _w_spec((8 * HID, 8 * HID)),
                  _w_spec((1, 8 * HID)), _w_spec((8 * HID, 8 * LAT)),
                  _w_spec((1, 8 * LAT)), _w_spec((1, 1, LAT)),
                  _w_spec((1, 1, LAT)), _w_spec((8 * LAT, 8 * HID)),
                  _w_spec((8 * LAT, 8 * HID))],
        out_shapes=[jax.ShapeDtypeStruct((EP8, 8 * HID), f32)] * 2,
        out_specs=[_row_spec(BP, 8 * HID)] * 2,
        name="tc_enc_edge", interpret=interpret,
    )(edp, k8(ee_["W1"]), t8(ee_["b1"]), k8(ee_["W2"]), t8(ee_["b2"]),
      ee_["g"].reshape(1, 1, LAT), ee_["bt"].reshape(1, 1, LAT),
      k8(A_e0), k8(A_e))

    zeros_nh = jnp.zeros((N_NODES, HID), f32)
    zeros_nl = jnp.zeros((N_NODES, LAT), f32)

    if interpret:
        def do_gather(eb, ec, ns_, nd_):
            s_ = lax.dynamic_slice_in_dim(src, eb, ec)
            d_ = lax.dynamic_slice_in_dim(dst, eb, ec)
            return jnp.take(ns_, s_, axis=0) + jnp.take(nd_, d_, axis=0)

        def do_count():
            s = jax.ops.segment_sum(jnp.ones((N_EDGES, HID), f32), dst,
                                    num_segments=N_NODES)
            return s, jnp.zeros_like(s)

        def do_scatter(eb, ec, pb, x):
            d_ = lax.dynamic_slice_in_dim(dst, eb, ec)
            x_ = lax.dynamic_slice_in_dim(x, pb, ec) if x.shape[0] != ec else x
            s = jax.ops.segment_sum(x_, d_, num_segments=N_NODES)
            return s, jnp.zeros_like(s)
    else:
        def do_gather(eb, ec, ns_, nd_):
            out = _sc_gather_k(eb, ec)(ns_, nd_, src, dst)
            return out[0] if isinstance(out, (list, tuple)) else out

        def do_count():
            return _sc_count_k()(dst, zeros_nh)

        def do_scatter(eb, ec, pb, x):
            return _sc_scatter_k(eb, ec, pb)(x, dst, zeros_nl)

    cntp = do_count()

    def edge_specs(e_base, pec_is_half):
        off = e_base // BE
        full = pl.BlockSpec((BP, 8 * HID), lambda i, o=off: (i + o, 0))
        half = _row_spec(BP, 8 * HID)
        return [full, half if pec_is_half else full, half]

    ew = [t8(pce["b1"]), k8(pce["W2"]), t8(pce["b2"]), g3ce, bt3ce]
    ew_specs = [_w_spec((1, 8 * HID)), _w_spec((8 * HID, 8 * LAT)),
                _w_spec((1, 8 * LAT)), _w_spec((1, 1, LAT)),
                _w_spec((1, 1, LAT))]

    Aeb = k8(A_e)
    pec_halves = None
    for t in range(CORE_STEPS - 1):
        hsums = [jnp.reshape(do_gather(eb, ec, ns, nd), (ec // 8, 8 * HID))
                 for (eb, ec) in CHUNKS]
        ens, aggs, pec_new = [], [], []
        for ci, (eb, ec) in enumerate(CHUNKS):
            pec_in = pec if pec_halves is None else pec_halves[ci]
            en, pc = _tc_call(
                _edge_step_body, ec // BE,
                in_specs=edge_specs(eb, pec_halves is not None) + ew_specs
                + [_w_spec((8 * LAT, 8 * HID))],
                out_shapes=[jax.ShapeDtypeStruct((ec, LAT), f32),
                            jax.ShapeDtypeStruct((ec // 8, 8 * HID), f32)],
                out_specs=[_row_spec(BE, LAT), _row_spec(BP, 8 * HID)],
                name=f"tc_edge_step{t}_{ci}", interpret=interpret,
            )(pe0, pec_in, hsums[ci], *ew, Aeb)
            ens.append(en)
            pec_new.append(pc)
            aggs.append(do_scatter(eb, ec, 0, en))
        pec_halves = pec_new

        ns, nd, pn = _tc_call(
            _node_step_body, N_NODES // BN,
            in_specs=[_row_spec(BN, HID), _row_spec(BN, LAT),
                      _row_spec(BN, LAT), _row_spec(BN, LAT),
                      _row_spec(BN, LAT), _row_spec(BN, HID),
                      _row_spec(BN, HID), _row_spec(BN, HID),
                      _row_spec(BN, HID), _row_spec(BN, HID),
                      w1h, _w_spec((HID, LAT)), w1l, w1l, w1l,
                      w16, w16, w16, w16],
            out_shapes=[jax.ShapeDtypeStruct((N_NODES, HID), f32)] * 3,
            out_specs=[_row_spec(BN, HID)] * 3,
            name=f"tc_node_step{t}", interpret=interpret,
        )(pn, aggs[0][0], aggs[0][1], aggs[1][0], aggs[1][1],
          cntp[0], cntp[1], f_s, f_d, f_n,
          r2(pcn["b1"]), pcn["W2"], r2(pcn["b2"]), r2(pcn["g"]),
          r2(pcn["bt"]), B_agg, A_vs, A_vd, B_v)

    hs3 = [jnp.reshape(do_gather(eb, ec, ns, nd), (ec // 8, 8 * HID))
           for (eb, ec) in CHUNKS]
    pec3 = jnp.concatenate(pec_halves, axis=0)
    nb0 = CHUNKS[0][1] // BE
    en3 = _tc_call(
        _edge_core_body, N_EDGES // BE,
        in_specs=[_row_spec(BP, 8 * HID), _row_spec(BP, 8 * HID),
                  pl.BlockSpec((BP, 8 * HID),
                               lambda i: (jnp.minimum(i, nb0 - 1), 0)),
                  pl.BlockSpec((BP, 8 * HID),
                               lambda i: (jnp.maximum(i - nb0, 0), 0))]
        + ew_specs,
        out_shapes=jax.ShapeDtypeStruct((N_EDGES, LAT), f32),
        out_specs=_row_spec(BE, LAT),
        name="tc_edge_core_last", interpret=interpret,
    )(pe0, pec3, hs3[0], hs3[1], *ew)

    aggs3 = [do_scatter(eb, ec, eb, en3) for (eb, ec) in CHUNKS]

    e_out = _tc_call(
        _edge_dec_body, N_EDGES // BE,
        in_specs=[_row_spec(BE, LAT),
                  _w_spec((LAT, HID)), w1h, _w_spec((HID, LAT)),
                  w1l, w1l, w1l, _w_spec((LAT, LAT)), w1l],
        out_shapes=jax.ShapeDtypeStruct((N_EDGES, LAT), f32),
        out_specs=_row_spec(BE, LAT),
        name="tc_edge_dec", interpret=interpret,
    )(en3, de_["W1"], r2(de_["b1"]), de_["W2"], r2(de_["b2"]),
      r2(de_["g"]), r2(de_["bt"]),
      params["dec_edge_out_W"], r2(params["dec_edge_out_b"]))

    v_out = _tc_call(
        _node_last_body, N_NODES // BN,
        in_specs=[_row_spec(BN, HID), _row_spec(BN, LAT),
                  _row_spec(BN, LAT), _row_spec(BN, LAT),
                  _row_spec(BN, LAT), _row_spec(BN, HID),
                  _row_spec(BN, HID),
                  w1h, _w_spec((HID, LAT)), w1l, w1l, w1l, w16,
                  _w_spec((LAT, HID)), w1h, _w_spec((HID, LAT)),
                  w1l, w1l, w1l, _w_spec((LAT, LAT)), w1l],
        out_shapes=jax.ShapeDtypeStruct((N_NODES, LAT), f32),
        out_specs=_row_spec(BN, LAT),
        name="tc_node_last", interpret=interpret,
    )(pn, aggs3[0][0], aggs3[0][1], aggs3[1][0], aggs3[1][1],
      cntp[0], cntp[1],
      r2(pcn["b1"]), pcn["W2"], r2(pcn["b2"]), r2(pcn["g"]),
      r2(pcn["bt"]), B_agg,
      dn_["W1"], r2(dn_["b1"]), dn_["W2"], r2(dn_["b2"]),
      r2(dn_["g"]), r2(dn_["bt"]),
      params["dec_node_out_W"], r2(params["dec_node_out_b"]))

    return (v_out, e_out)


def kernel(vdata, edata, connectivity, cdata, metadata, params):
    return _forward_impl(vdata, edata, connectivity, params)


# speedup vs baseline: 1.8389x; 1.0100x over previous
import functools
import jax
import jax.numpy as jnp
from jax import lax
from jax.experimental import pallas as pl
from jax.experimental.pallas import tpu as pltpu
from jax.experimental.pallas import tpu_sc as plsc

N_NODES = 10000
N_EDGES = 160000
LAT = 128
HID = 16
CORE_STEPS = 3

BN = 2000
BE = 8000
BP = BE // 8
EP8 = N_EDGES // 8
NW = 32
EPW = N_EDGES // NW
NPT = N_NODES // 16
CH = 200
NCH = EPW // CH

_EPS = 1e-5


def _ln(h, g, bt):
    mu = jnp.mean(h, axis=-1, keepdims=True)
    var = jnp.mean((h - mu) * (h - mu), axis=-1, keepdims=True)
    return (h - mu) * lax.rsqrt(var + _EPS) * g + bt


def _dot(a, b):
    return jnp.dot(a, b, preferred_element_type=jnp.float32)


def _mlp_tail(h1pre, W2, b2, g, bt):
    h = jnp.maximum(h1pre, 0.0)
    h = jnp.maximum(_dot(h, W2) + b2, 0.0)
    return _ln(h, g, bt)


def _row_spec(b, d):
    return pl.BlockSpec((b, d), lambda i: (i, 0))


def _w_spec(shape):
    return pl.BlockSpec(shape, lambda i: tuple(0 for _ in shape))


def _enc_node_body(x, W1, b1, W2, b2, g, bt, Avs, Avd, Bv, Av0s, Av0d, Bv0,
                   ns_o, nd_o, pn_o, fs_o, fd_o, fn_o):
    h1 = _dot(x[...], W1[...]) + b1[...]
    v0 = _mlp_tail(h1, W2[...], b2[...], g[...], bt[...])
    fs = _dot(v0, Av0s[...])
    fd = _dot(v0, Av0d[...])
    fn = _dot(v0, Bv0[...])
    fs_o[...] = fs
    fd_o[...] = fd
    fn_o[...] = fn
    ns_o[...] = fs + _dot(v0, Avs[...])
    nd_o[...] = fd + _dot(v0, Avd[...])
    pn_o[...] = fn + _dot(v0, Bv[...])


def _enc_edge_body(xp, W1b, b1t, W2b, b2t, g3, bt3, A0b, A1b, pe0_o, pec_o):
    h1p = jnp.maximum(_dot(xp[...], W1b[...]) + b1t[...], 0.0)
    enw = jnp.maximum(_dot(h1p, W2b[...]) + b2t[...], 0.0)
    e3 = jnp.reshape(enw, (BP, 8, LAT))
    mu = jnp.mean(e3, axis=-1, keepdims=True)
    var = jnp.mean((e3 - mu) * (e3 - mu), axis=-1, keepdims=True)
    e3 = (e3 - mu) * lax.rsqrt(var + _EPS) * g3[...] + bt3[...]
    ef = jnp.reshape(e3, (BP, 8 * LAT))
    pe0_o[...] = _dot(ef, A0b[...])
    pec_o[...] = _dot(ef, A1b[...])


def _edge_step_body(pe0, pec, hs, b1t, W2b, b2t, g3, bt3, Aeb,
                    en_o, pec_o):
    h1p = jnp.maximum(pe0[...] + pec[...] + hs[...] + b1t[...], 0.0)
    enw = jnp.maximum(_dot(h1p, W2b[...]) + b2t[...], 0.0)
    e3 = jnp.reshape(enw, (BP, 8, LAT))
    mu = jnp.mean(e3, axis=-1, keepdims=True)
    var = jnp.mean((e3 - mu) * (e3 - mu), axis=-1, keepdims=True)
    e3 = (e3 - mu) * lax.rsqrt(var + _EPS) * g3[...] + bt3[...]
    en_o[...] = jnp.reshape(e3, (BE, LAT))
    pec_o[...] = _dot(jnp.reshape(e3, (BP, 8 * LAT)), Aeb[...])


def _edge_core_body(pe0, pec, hs0, hs1, b1t, W2b, b2t, g3, bt3, en_o):
    i = pl.program_id(0)
    nb0 = CHUNKS[0][1] // BE
    hs = jnp.where(i < nb0, hs0[...], hs1[...])
    h1p = jnp.maximum(pe0[...] + pec[...] + hs + b1t[...], 0.0)
    enw = jnp.maximum(_dot(h1p, W2b[...]) + b2t[...], 0.0)
    e3 = jnp.reshape(enw, (BP, 8, LAT))
    mu = jnp.mean(e3, axis=-1, keepdims=True)
    var = jnp.mean((e3 - mu) * (e3 - mu), axis=-1, keepdims=True)
    e3 = (e3 - mu) * lax.rsqrt(var + _EPS) * g3[...] + bt3[...]
    en_o[...] = jnp.reshape(e3, (BE, LAT))


def _edge_dec_body(en, dW1, db1, dW2, db2, dg, dbt, oW, ob, eout_o):
    d1 = _dot(en[...], dW1[...]) + db1[...]
    dec = _mlp_tail(d1, dW2[...], db2[...], dg[...], dbt[...])
    eout_o[...] = _dot(dec, oW[...]) + ob[...]


def _node_step_body(pn, a0, a1, a2, a3, c0, c1, fs, fd, fn, b1, W2, b2, g, bt,
                    Bagg, Avs, Avd, Bv, ns_o, nd_o, pn_o):
    cm = jnp.maximum(c0[...] + c1[...], 1.0)[:, 0:1]
    agg = (a0[...] + a1[...] + a2[...] + a3[...]) / cm
    h1 = pn[...] + _dot(agg, Bagg[...]) + b1[...]
    vn = _mlp_tail(h1, W2[...], b2[...], g[...], bt[...])
    ns_o[...] = fs[...] + _dot(vn, Avs[...])
    nd_o[...] = fd[...] + _dot(vn, Avd[...])
    pn_o[...] = fn[...] + _dot(vn, Bv[...])


def _node_last_body(pn, a0, a1, a2, a3, c0, c1, b1, W2, b2, g, bt, Bagg,
                    dW1, db1, dW2, db2, dg, dbt, oW, ob, vout_o):
    cm = jnp.maximum(c0[...] + c1[...], 1.0)[:, 0:1]
    agg = (a0[...] + a1[...] + a2[...] + a3[...]) / cm
    h1 = pn[...] + _dot(agg, Bagg[...]) + b1[...]
    vn = _mlp_tail(h1, W2[...], b2[...], g[...], bt[...])
    d1 = _dot(vn, dW1[...]) + db1[...]
    dec = _mlp_tail(d1, dW2[...], db2[...], dg[...], dbt[...])
    vout_o[...] = _dot(dec, oW[...]) + ob[...]


def _tc_call(body, grid, in_specs, out_shapes, out_specs, name,
             interpret=False):
    return pl.pallas_call(
        body,
        grid=(grid,),
        in_specs=in_specs,
        out_specs=out_specs,
        out_shape=out_shapes,
        interpret=interpret,
        name=name,
    )


_MESHF = plsc.VectorSubcoreMesh
_CPF = pltpu.CompilerParams


@functools.cache
def _sc_gather_k(e_base, e_cnt):
    epw = e_cnt // NW
    mesh = _MESHF(core_axis_name="c", subcore_axis_name="s")

    @functools.partial(
        pl.kernel,
        out_type=[jax.ShapeDtypeStruct((e_cnt, HID), jnp.float32)],
        mesh=mesh,
        compiler_params=_CPF(use_tc_tiling_on_sc=False),
        scratch_types=[pltpu.VMEM((epw,), jnp.int32),
                       pltpu.VMEM((epw,), jnp.int32),
                       pltpu.VMEM((epw, HID), jnp.float32),
                       pltpu.VMEM((epw, HID), jnp.float32),
                       pltpu.SemaphoreType.DMA],
        name=f"sc_gather_{e_base}_{e_cnt}",
    )
    def sc_gather(ns_h, nd_h, src_h, dst_h, hs_h, idx_s, idx_d, rs_v, rd_v,
                  sem):
        wid = lax.axis_index("s") * 2 + lax.axis_index("c")
        lsl = pl.ds(wid * epw, epw)
        gsl = pl.ds(e_base + wid * epw, epw)
        pltpu.sync_copy(src_h.at[gsl], idx_s)
        pltpu.sync_copy(dst_h.at[gsl], idx_d)
        cps = pltpu.async_copy(ns_h.at[idx_s], rs_v, sem)
        cpd = pltpu.async_copy(nd_h.at[idx_d], rd_v, sem)
        cps.wait()
        cpd.wait()

        def add4(i, _):
            for k in range(4):
                r = i * 4 + k
                rs_v[r, :] = rs_v[r, :] + rd_v[r, :]
            return 0

        lax.fori_loop(0, epw // 4, add4, 0, unroll=False)
        pltpu.sync_copy(rs_v, hs_h.at[lsl])

    return sc_gather


@functools.cache
def _sc_count_k():
    mesh = _MESHF(core_axis_name="c", subcore_axis_name="s")

    @functools.partial(
        pl.kernel,
        out_type=[jax.ShapeDtypeStruct((N_NODES, HID), jnp.float32),
                  jax.ShapeDtypeStruct((N_NODES, HID), jnp.float32)],
        mesh=mesh,
        compiler_params=_CPF(use_tc_tiling_on_sc=False),
        scratch_types=[pltpu.VMEM((EPW,), jnp.int32),
                       pltpu.VMEM((EPW, HID), jnp.float32),
                       pltpu.VMEM_SHARED((N_NODES, HID), jnp.float32)],
        name="sc_count",
    )
    def sc_count(dst_h, zeros_h, out0_h, out1_h, idx_v, pa_v, acc):
        sid = lax.axis_index("s")
        cid = lax.axis_index("c")
        base = (sid * 2 + cid) * EPW
        nsl = pl.ds(sid * NPT, NPT)
        pltpu.sync_copy(zeros_h.at[nsl], acc.at[nsl])

        one = jnp.ones((HID,), jnp.float32)

        def fill4(i, _):
            for k in range(4):
                pa_v[i * 4 + k, :] = one
            return 0

        lax.fori_loop(0, EPW // 4, fill4, 0, unroll=False)
        plsc.subcore_barrier()
        pltpu.sync_copy(dst_h.at[pl.ds(base, EPW)], idx_v)
        pltpu.sync_copy(pa_v, acc.at[idx_v], add=True)
        plsc.subcore_barrier()

        @pl.when(cid == 0)
        def _():
            pltpu.sync_copy(acc.at[nsl], out0_h.at[nsl])

        @pl.when(cid == 1)
        def _():
            pltpu.sync_copy(acc.at[nsl], out1_h.at[nsl])

    return sc_count


@functools.cache
def _sc_scatter_k(e_base, e_cnt, pa_base):
    epw = e_cnt // NW
    ch = 120 if epw % 120 == 0 else 80
    nch = epw // ch
    mesh = _MESHF(core_axis_name="c", subcore_axis_name="s")

    @functools.partial(
        pl.kernel,
        out_type=[jax.ShapeDtypeStruct((N_NODES, LAT), jnp.float32),
                  jax.ShapeDtypeStruct((N_NODES, LAT), jnp.float32)],
        mesh=mesh,
        compiler_params=_CPF(use_tc_tiling_on_sc=False),
        scratch_types=[pltpu.VMEM((nch, ch), jnp.int32),
                       pltpu.VMEM((2, ch, LAT), jnp.float32),
                       pltpu.VMEM_SHARED((N_NODES, LAT), jnp.float32),
                       pltpu.SemaphoreType.DMA,
                       pltpu.SemaphoreType.DMA,
                       pltpu.SemaphoreType.DMA,
                       pltpu.SemaphoreType.DMA,
                       pltpu.SemaphoreType.DMA],
        name=f"sc_scatter_{e_base}_{e_cnt}",
    )
    def sc_scatter(pa_h, dst_h, zeros_h, out0_h, out1_h, idx2, pav, acc,
                   semi, seml0, seml1, sems0, sems1):
        sid = lax.axis_index("s")
        cid = lax.axis_index("c")
        lbase = (sid * 2 + cid) * epw
        gbase = e_base + lbase
        pbase = pa_base + lbase
        nsl = pl.ds(sid * NPT, NPT)
        icps = [pltpu.async_copy(dst_h.at[pl.ds(gbase + j * ch, ch)],
                                 idx2.at[j], semi) for j in range(nch)]
        pltpu.sync_copy(zeros_h.at[nsl], acc.at[nsl])
        for c in icps:
            c.wait()
        plsc.subcore_barrier()
        seml = [seml0, seml1]
        lds = [None] * nch
        lds[0] = pltpu.async_copy(pa_h.at[pl.ds(pbase, ch)], pav.at[0],
                                  seml[0])
        for j in range(nch):
            lds[j].wait()
            if j + 1 < nch:
                lds[j + 1] = pltpu.async_copy(
                    pa_h.at[pl.ds(pbase + (j + 1) * ch, ch)],
                    pav.at[(j + 1) % 2], seml[(j + 1) % 2])
            pltpu.sync_copy(pav.at[j % 2], acc.at[idx2.at[j]], add=True)
        plsc.subcore_barrier()

        @pl.when(cid == 0)
        def _():
            pltpu.sync_copy(acc.at[nsl], out0_h.at[nsl])

        @pl.when(cid == 1)
        def _():
            pltpu.sync_copy(acc.at[nsl], out1_h.at[nsl])

    return sc_scatter


CHUNKS = ((0, 96000), (96000, 64000))


def _forward_impl(vdata, edata, connectivity, params, interpret=False):
    f32 = jnp.float32
    src = connectivity[0]
    dst = connectivity[1]

    pce, pcn = params["core_edge"], params["core_node"]
    W1ce, W1cn = pce["W1"], pcn["W1"]
    A_e0, A_e = W1ce[0:LAT], W1ce[LAT:2 * LAT]
    A_v0s, A_vs = W1ce[2 * LAT:3 * LAT], W1ce[3 * LAT:4 * LAT]
    A_v0d, A_vd = W1ce[4 * LAT:5 * LAT], W1ce[5 * LAT:6 * LAT]
    B_v0, B_v, B_agg = W1cn[0:LAT], W1cn[LAT:2 * LAT], W1cn[2 * LAT:3 * LAT]

    def r2(x):
        return x.reshape(1, -1)

    en_, ee_ = params["enc_node"], params["enc_edge"]
    dn_, de_ = params["dec_node"], params["dec_edge"]

    w16 = _w_spec((LAT, HID))
    w1h = _w_spec((1, HID))
    w1l = _w_spec((1, LAT))

    ns, nd, pn, f_s, f_d, f_n = _tc_call(
        _enc_node_body, N_NODES // BN,
        in_specs=[_row_spec(BN, LAT), _w_spec((LAT, HID)), w1h,
                  _w_spec((HID, LAT)), w1l, w1l, w1l,
                  w16, w16, w16, w16, w16, w16],
        out_shapes=[jax.ShapeDtypeStruct((N_NODES, HID), f32)] * 6,
        out_specs=[_row_spec(BN, HID)] * 6,
        name="tc_enc_node", interpret=interpret,
    )(vdata, en_["W1"], r2(en_["b1"]), en_["W2"], r2(en_["b2"]),
      r2(en_["g"]), r2(en_["bt"]),
      A_vs, A_vd, B_v, A_v0s, A_v0d, B_v0)

    eye8 = jnp.eye(8, dtype=f32)
    k8 = lambda w: jnp.kron(eye8, w)
    t8 = lambda b: jnp.tile(b, 8).reshape(1, -1)
    g3ce = pce["g"].reshape(1, 1, LAT)
    bt3ce = pce["bt"].reshape(1, 1, LAT)
    edp = jnp.reshape(edata, (EP8, 8 * HID))
    wp = _w_spec((BP, 8 * HID))
    wpl = _w_spec((BP, LAT))

    pe0, pec = _tc_call(
        _enc_edge_body, N_EDGES // BE,
        in_specs=[_row_spec(BP, 8 * HID), _w_spec((8 * HID, 8 * HID)),
                  _w_spec((1, 8 * HID)), _w_spec((8 * HID, 8 * LAT)),
                  _w_spec((1, 8 * LAT)), _w_spec((1, 1, LAT)),
                  _w_spec((1, 1, LAT)), _w_spec((8 * LAT, 8 * HID)),
                  _w_spec((8 * LAT, 8 * HID))],
        out_shapes=[jax.ShapeDtypeStruct((EP8, 8 * HID), f32)] * 2,
        out_specs=[_row_spec(BP, 8 * HID)] * 2,
        name="tc_enc_edge", interpret=interpret,
    )(edp, k8(ee_["W1"]), t8(ee_["b1"]), k8(ee_["W2"]), t8(ee_["b2"]),
      ee_["g"].reshape(1, 1, LAT), ee_["bt"].reshape(1, 1, LAT),
      k8(A_e0), k8(A_e))

    zeros_nh = jnp.zeros((N_NODES, HID), f32)
    zeros_nl = jnp.zeros((N_NODES, LAT), f32)

    if interpret:
        def do_gather(eb, ec, ns_, nd_):
            s_ = lax.dynamic_slice_in_dim(src, eb, ec)
            d_ = lax.dynamic_slice_in_dim(dst, eb, ec)
            return jnp.take(ns_, s_, axis=0) + jnp.take(nd_, d_, axis=0)

        def do_count():
            s = jax.ops.segment_sum(jnp.ones((N_EDGES, HID), f32), dst,
                                    num_segments=N_NODES)
            return s, jnp.zeros_like(s)

        def do_scatter(eb, ec, pb, x):
            d_ = lax.dynamic_slice_in_dim(dst, eb, ec)
            x_ = lax.dynamic_slice_in_dim(x, pb, ec) if x.shape[0] != ec else x
            s = jax.ops.segment_sum(x_, d_, num_segments=N_NODES)
            return s, jnp.zeros_like(s)
    else:
        def do_gather(eb, ec, ns_, nd_):
            out = _sc_gather_k(eb, ec)(ns_, nd_, src, dst)
            return out[0] if isinstance(out, (list, tuple)) else out

        def do_count():
            return _sc_count_k()(dst, zeros_nh)

        def do_scatter(eb, ec, pb, x):
            return _sc_scatter_k(eb, ec, pb)(x, dst, zeros_nl)

    cntp = do_count()

    def edge_specs(e_base, pec_is_half):
        off = e_base // BE
        full = pl.BlockSpec((BP, 8 * HID), lambda i, o=off: (i + o, 0))
        half = _row_spec(BP, 8 * HID)
        return [full, half if pec_is_half else full, half]

    ew = [t8(pce["b1"]), k8(pce["W2"]), t8(pce["b2"]), g3ce, bt3ce]
    ew_specs = [_w_spec((1, 8 * HID)), _w_spec((8 * HID, 8 * LAT)),
                _w_spec((1, 8 * LAT)), _w_spec((1, 1, LAT)),
                _w_spec((1, 1, LAT))]

    Aeb = k8(A_e)
    pec_halves = None
    for t in range(CORE_STEPS - 1):
        hsums = [jnp.reshape(do_gather(eb, ec, ns, nd), (ec // 8, 8 * HID))
                 for (eb, ec) in CHUNKS]
        ens, aggs, pec_new = [], [], []
        for ci, (eb, ec) in enumerate(CHUNKS):
            pec_in = pec if pec_halves is None else pec_halves[ci]
            en, pc = _tc_call(
                _edge_step_body, ec // BE,
                in_specs=edge_specs(eb, pec_halves is not None) + ew_specs
                + [_w_spec((8 * LAT, 8 * HID))],
                out_shapes=[jax.ShapeDtypeStruct((ec, LAT), f32),
                            jax.ShapeDtypeStruct((ec // 8, 8 * HID), f32)],
                out_specs=[_row_spec(BE, LAT), _row_spec(BP, 8 * HID)],
                name=f"tc_edge_step{t}_{ci}", interpret=interpret,
            )(pe0, pec_in, hsums[ci], *ew, Aeb)
            ens.append(en)
            pec_new.append(pc)
            aggs.append(do_scatter(eb, ec, 0, en))
        pec_halves = pec_new

        ns, nd, pn = _tc_call(
            _node_step_body, N_NODES // BN,
            in_specs=[_row_spec(BN, HID), _row_spec(BN, LAT),
                      _row_spec(BN, LAT), _row_spec(BN, LAT),
                      _row_spec(BN, LAT), _row_spec(BN, HID),
                      _row_spec(BN, HID), _row_spec(BN, HID),
                      _row_spec(BN, HID), _row_spec(BN, HID),
                      w1h, _w_spec((HID, LAT)), w1l, w1l, w1l,
                      w16, w16, w16, w16],
            out_shapes=[jax.ShapeDtypeStruct((N_NODES, HID), f32)] * 3,
            out_specs=[_row_spec(BN, HID)] * 3,
            name=f"tc_node_step{t}", interpret=interpret,
        )(pn, aggs[0][0], aggs[0][1], aggs[1][0], aggs[1][1],
          cntp[0], cntp[1], f_s, f_d, f_n,
          r2(pcn["b1"]), pcn["W2"], r2(pcn["b2"]), r2(pcn["g"]),
          r2(pcn["bt"]), B_agg, A_vs, A_vd, B_v)

    hs3 = [jnp.reshape(do_gather(eb, ec, ns, nd), (ec // 8, 8 * HID))
           for (eb, ec) in CHUNKS]
    pec3 = jnp.concatenate(pec_halves, axis=0)
    nb0 = CHUNKS[0][1] // BE
    en3 = _tc_call(
        _edge_core_body, N_EDGES // BE,
        in_specs=[_row_spec(BP, 8 * HID), _row_spec(BP, 8 * HID),
                  pl.BlockSpec((BP, 8 * HID),
                               lambda i: (jnp.minimum(i, nb0 - 1), 0)),
                  pl.BlockSpec((BP, 8 * HID),
                               lambda i: (jnp.maximum(i - nb0, 0), 0))]
        + ew_specs,
        out_shapes=jax.ShapeDtypeStruct((N_EDGES, LAT), f32),
        out_specs=_row_spec(BE, LAT),
        name="tc_edge_core_last", interpret=interpret,
    )(pe0, pec3, hs3[0], hs3[1], *ew)

    aggs3 = [do_scatter(eb, ec, eb, en3) for (eb, ec) in CHUNKS]

    e_out = _tc_call(
        _edge_dec_body, N_EDGES // BE,
        in_specs=[_row_spec(BE, LAT),
                  _w_spec((LAT, HID)), w1h, _w_spec((HID, LAT)),
                  w1l, w1l, w1l, _w_spec((LAT, LAT)), w1l],
        out_shapes=jax.ShapeDtypeStruct((N_EDGES, LAT), f32),
        out_specs=_row_spec(BE, LAT),
        name="tc_edge_dec", interpret=interpret,
    )(en3, de_["W1"], r2(de_["b1"]), de_["W2"], r2(de_["b2"]),
      r2(de_["g"]), r2(de_["bt"]),
      params["dec_edge_out_W"], r2(params["dec_edge_out_b"]))

    v_out = _tc_call(
        _node_last_body, N_NODES // BN,
        in_specs=[_row_spec(BN, HID), _row_spec(BN, LAT),
                  _row_spec(BN, LAT), _row_spec(BN, LAT),
                  _row_spec(BN, LAT), _row_spec(BN, HID),
                  _row_spec(BN, HID),
                  w1h, _w_spec((HID, LAT)), w1l, w1l, w1l, w16,
                  _w_spec((LAT, HID)), w1h, _w_spec((HID, LAT)),
                  w1l, w1l, w1l, _w_spec((LAT, LAT)), w1l],
        out_shapes=jax.ShapeDtypeStruct((N_NODES, LAT), f32),
        out_specs=_row_spec(BN, LAT),
        name="tc_node_last", interpret=interpret,
    )(pn, aggs3[0][0], aggs3[0][1], aggs3[1][0], aggs3[1][1],
      cntp[0], cntp[1],
      r2(pcn["b1"]), pcn["W2"], r2(pcn["b2"]), r2(pcn["g"]),
      r2(pcn["bt"]), B_agg,
      dn_["W1"], r2(dn_["b1"]), dn_["W2"], r2(dn_["b2"]),
      r2(dn_["g"]), r2(dn_["bt"]),
      params["dec_node_out_W"], r2(params["dec_node_out_b"]))

    return (v_out, e_out)


def kernel(vdata, edata, connectivity, cdata, metadata, params):
    return _forward_impl(vdata, edata, connectivity, params)


# speedup vs baseline: 1.8391x; 1.0001x over previous
import functools
import jax
import jax.numpy as jnp
from jax import lax
from jax.experimental import pallas as pl
from jax.experimental.pallas import tpu as pltpu
from jax.experimental.pallas import tpu_sc as plsc

N_NODES = 10000
N_EDGES = 160000
LAT = 128
HID = 16
CORE_STEPS = 3

BN = 2000
BE = 8000
BP = BE // 8
EP8 = N_EDGES // 8
NW = 32
EPW = N_EDGES // NW
NPT = N_NODES // 16

_EPS = 1e-5


def _ln(h, g, bt):
    mu = jnp.mean(h, axis=-1, keepdims=True)
    var = jnp.mean((h - mu) * (h - mu), axis=-1, keepdims=True)
    return (h - mu) * lax.rsqrt(var + _EPS) * g + bt


def _dot(a, b):
    return jnp.dot(a, b, preferred_element_type=jnp.float32)


def _mlp_tail(h1pre, W2, b2, g, bt):
    h = jnp.maximum(h1pre, 0.0)
    h = jnp.maximum(_dot(h, W2) + b2, 0.0)
    return _ln(h, g, bt)


def _row_spec(b, d):
    return pl.BlockSpec((b, d), lambda i: (i, 0))


def _w_spec(shape):
    return pl.BlockSpec(shape, lambda i: tuple(0 for _ in shape))


def _enc_node_body(x, W1, b1, W2, b2, g, bt, Avs, Avd, Bv, Av0s, Av0d, Bv0,
                   ns_o, nd_o, pn_o, fs_o, fd_o, fn_o):
    h1 = _dot(x[...], W1[...]) + b1[...]
    v0 = _mlp_tail(h1, W2[...], b2[...], g[...], bt[...])
    fs = _dot(v0, Av0s[...])
    fd = _dot(v0, Av0d[...])
    fn = _dot(v0, Bv0[...])
    fs_o[...] = fs
    fd_o[...] = fd
    fn_o[...] = fn
    ns_o[...] = fs + _dot(v0, Avs[...])
    nd_o[...] = fd + _dot(v0, Avd[...])
    pn_o[...] = fn + _dot(v0, Bv[...])


def _enc_edge_body(xp, W1b, b1t, W2b, b2t, g3, bt3, A0b, A1b, pe0_o, pec_o):
    h1p = jnp.maximum(_dot(xp[...], W1b[...]) + b1t[...], 0.0)
    enw = jnp.maximum(_dot(h1p, W2b[...]) + b2t[...], 0.0)
    e3 = jnp.reshape(enw, (BP, 8, LAT))
    mu = jnp.mean(e3, axis=-1, keepdims=True)
    var = jnp.mean((e3 - mu) * (e3 - mu), axis=-1, keepdims=True)
    e3 = (e3 - mu) * lax.rsqrt(var + _EPS) * g3[...] + bt3[...]
    ef = jnp.reshape(e3, (BP, 8 * LAT))
    pe0_o[...] = _dot(ef, A0b[...])
    pec_o[...] = _dot(ef, A1b[...])


def _edge_step_body(pe0, pec, hs, b1t, W2b, b2t, g3, bt3, Aeb,
                    en_o, pec_o):
    h1p = jnp.maximum(pe0[...] + pec[...] + hs[...] + b1t[...], 0.0)
    enw = jnp.maximum(_dot(h1p, W2b[...]) + b2t[...], 0.0)
    e3 = jnp.reshape(enw, (BP, 8, LAT))
    mu = jnp.mean(e3, axis=-1, keepdims=True)
    var = jnp.mean((e3 - mu) * (e3 - mu), axis=-1, keepdims=True)
    e3 = (e3 - mu) * lax.rsqrt(var + _EPS) * g3[...] + bt3[...]
    en_o[...] = jnp.reshape(e3, (BE, LAT))
    pec_o[...] = _dot(jnp.reshape(e3, (BP, 8 * LAT)), Aeb[...])


def _edge_core_body(pe0, pec, hs0, hs1, b1t, W2b, b2t, g3, bt3, en_o):
    i = pl.program_id(0)
    nb0 = CHUNKS[0][1] // BE
    hs = jnp.where(i < nb0, hs0[...], hs1[...])
    h1p = jnp.maximum(pe0[...] + pec[...] + hs + b1t[...], 0.0)
    enw = jnp.maximum(_dot(h1p, W2b[...]) + b2t[...], 0.0)
    e3 = jnp.reshape(enw, (BP, 8, LAT))
    mu = jnp.mean(e3, axis=-1, keepdims=True)
    var = jnp.mean((e3 - mu) * (e3 - mu), axis=-1, keepdims=True)
    e3 = (e3 - mu) * lax.rsqrt(var + _EPS) * g3[...] + bt3[...]
    en_o[...] = jnp.reshape(e3, (BE, LAT))


def _edge_dec_body(en, dW1, db1, dW2, db2, dg, dbt, oW, ob, eout_o):
    d1 = _dot(en[...], dW1[...]) + db1[...]
    dec = _mlp_tail(d1, dW2[...], db2[...], dg[...], dbt[...])
    eout_o[...] = _dot(dec, oW[...]) + ob[...]


def _node_step_body(pn, a0, a1, a2, a3, c0, c1, fs, fd, fn, b1, W2, b2, g, bt,
                    Bagg, Avs, Avd, Bv, ns_o, nd_o, pn_o):
    cm = jnp.maximum(c0[...] + c1[...], 1.0)[:, 0:1]
    agg = (a0[...] + a1[...] + a2[...] + a3[...]) / cm
    h1 = pn[...] + _dot(agg, Bagg[...]) + b1[...]
    vn = _mlp_tail(h1, W2[...], b2[...], g[...], bt[...])
    ns_o[...] = fs[...] + _dot(vn, Avs[...])
    nd_o[...] = fd[...] + _dot(vn, Avd[...])
    pn_o[...] = fn[...] + _dot(vn, Bv[...])


def _node_last_body(pn, a0, a1, a2, a3, c0, c1, b1, W2, b2, g, bt, Bagg,
                    dW1, db1, dW2, db2, dg, dbt, oW, ob, vout_o):
    cm = jnp.maximum(c0[...] + c1[...], 1.0)[:, 0:1]
    agg = (a0[...] + a1[...] + a2[...] + a3[...]) / cm
    h1 = pn[...] + _dot(agg, Bagg[...]) + b1[...]
    vn = _mlp_tail(h1, W2[...], b2[...], g[...], bt[...])
    d1 = _dot(vn, dW1[...]) + db1[...]
    dec = _mlp_tail(d1, dW2[...], db2[...], dg[...], dbt[...])
    vout_o[...] = _dot(dec, oW[...]) + ob[...]


def _tc_call(body, grid, in_specs, out_shapes, out_specs, name):
    return pl.pallas_call(
        body,
        grid=(grid,),
        in_specs=in_specs,
        out_specs=out_specs,
        out_shape=out_shapes,
        name=name,
    )


_MESHF = plsc.VectorSubcoreMesh
_CPF = pltpu.CompilerParams


@functools.cache
def _sc_gather_k(e_base, e_cnt):
    epw = e_cnt // NW
    mesh = _MESHF(core_axis_name="c", subcore_axis_name="s")

    @functools.partial(
        pl.kernel,
        out_type=[jax.ShapeDtypeStruct((e_cnt, HID), jnp.float32)],
        mesh=mesh,
        compiler_params=_CPF(use_tc_tiling_on_sc=False),
        scratch_types=[pltpu.VMEM((epw,), jnp.int32),
                       pltpu.VMEM((epw,), jnp.int32),
                       pltpu.VMEM((epw, HID), jnp.float32),
                       pltpu.VMEM((epw, HID), jnp.float32),
                       pltpu.SemaphoreType.DMA],
        name=f"sc_gather_{e_base}_{e_cnt}",
    )
    def sc_gather(ns_h, nd_h, src_h, dst_h, hs_h, idx_s, idx_d, rs_v, rd_v,
                  sem):
        wid = lax.axis_index("s") * 2 + lax.axis_index("c")
        lsl = pl.ds(wid * epw, epw)
        gsl = pl.ds(e_base + wid * epw, epw)
        pltpu.sync_copy(src_h.at[gsl], idx_s)
        pltpu.sync_copy(dst_h.at[gsl], idx_d)
        cps = pltpu.async_copy(ns_h.at[idx_s], rs_v, sem)
        cpd = pltpu.async_copy(nd_h.at[idx_d], rd_v, sem)
        cps.wait()
        cpd.wait()

        def add4(i, _):
            for k in range(4):
                r = i * 4 + k
                rs_v[r, :] = rs_v[r, :] + rd_v[r, :]
            return 0

        lax.fori_loop(0, epw // 4, add4, 0, unroll=False)
        pltpu.sync_copy(rs_v, hs_h.at[lsl])

    return sc_gather


@functools.cache
def _sc_count_k():
    mesh = _MESHF(core_axis_name="c", subcore_axis_name="s")

    @functools.partial(
        pl.kernel,
        out_type=[jax.ShapeDtypeStruct((N_NODES, HID), jnp.float32),
                  jax.ShapeDtypeStruct((N_NODES, HID), jnp.float32)],
        mesh=mesh,
        compiler_params=_CPF(use_tc_tiling_on_sc=False),
        scratch_types=[pltpu.VMEM((EPW,), jnp.int32),
                       pltpu.VMEM((EPW, HID), jnp.float32),
                       pltpu.VMEM_SHARED((N_NODES, HID), jnp.float32)],
        name="sc_count",
    )
    def sc_count(dst_h, zeros_h, out0_h, out1_h, idx_v, pa_v, acc):
        sid = lax.axis_index("s")
        cid = lax.axis_index("c")
        base = (sid * 2 + cid) * EPW
        nsl = pl.ds(sid * NPT, NPT)
        pltpu.sync_copy(zeros_h.at[nsl], acc.at[nsl])

        one = jnp.ones((HID,), jnp.float32)

        def fill4(i, _):
            for k in range(4):
                pa_v[i * 4 + k, :] = one
            return 0

        lax.fori_loop(0, EPW // 4, fill4, 0, unroll=False)
        plsc.subcore_barrier()
        pltpu.sync_copy(dst_h.at[pl.ds(base, EPW)], idx_v)
        pltpu.sync_copy(pa_v, acc.at[idx_v], add=True)
        plsc.subcore_barrier()

        @pl.when(cid == 0)
        def _():
            pltpu.sync_copy(acc.at[nsl], out0_h.at[nsl])

        @pl.when(cid == 1)
        def _():
            pltpu.sync_copy(acc.at[nsl], out1_h.at[nsl])

    return sc_count


@functools.cache
def _sc_scatter_k(e_base, e_cnt, pa_base):
    epw = e_cnt // NW
    ch = 120 if epw % 120 == 0 else 80
    nch = epw // ch
    mesh = _MESHF(core_axis_name="c", subcore_axis_name="s")

    @functools.partial(
        pl.kernel,
        out_type=[jax.ShapeDtypeStruct((N_NODES, LAT), jnp.float32),
                  jax.ShapeDtypeStruct((N_NODES, LAT), jnp.float32)],
        mesh=mesh,
        compiler_params=_CPF(use_tc_tiling_on_sc=False),
        scratch_types=[pltpu.VMEM((nch, ch), jnp.int32),
                       pltpu.VMEM((2, ch, LAT), jnp.float32),
                       pltpu.VMEM_SHARED((N_NODES, LAT), jnp.float32),
                       pltpu.SemaphoreType.DMA,
                       pltpu.SemaphoreType.DMA,
                       pltpu.SemaphoreType.DMA,
                       pltpu.SemaphoreType.DMA,
                       pltpu.SemaphoreType.DMA],
        name=f"sc_scatter_{e_base}_{e_cnt}",
    )
    def sc_scatter(pa_h, dst_h, zeros_h, out0_h, out1_h, idx2, pav, acc,
                   semi, seml0, seml1, sems0, sems1):
        sid = lax.axis_index("s")
        cid = lax.axis_index("c")
        lbase = (sid * 2 + cid) * epw
        gbase = e_base + lbase
        pbase = pa_base + lbase
        nsl = pl.ds(sid * NPT, NPT)
        icps = [pltpu.async_copy(dst_h.at[pl.ds(gbase + j * ch, ch)],
                                 idx2.at[j], semi) for j in range(nch)]
        pltpu.sync_copy(zeros_h.at[nsl], acc.at[nsl])
        for c in icps:
            c.wait()
        plsc.subcore_barrier()
        seml = [seml0, seml1]
        lds = [None] * nch
        lds[0] = pltpu.async_copy(pa_h.at[pl.ds(pbase, ch)], pav.at[0],
                                  seml[0])
        for j in range(nch):
            lds[j].wait()
            if j + 1 < nch:
                lds[j + 1] = pltpu.async_copy(
                    pa_h.at[pl.ds(pbase + (j + 1) * ch, ch)],
                    pav.at[(j + 1) % 2], seml[(j + 1) % 2])
            pltpu.sync_copy(pav.at[j % 2], acc.at[idx2.at[j]], add=True)
        plsc.subcore_barrier()

        @pl.when(cid == 0)
        def _():
            pltpu.sync_copy(acc.at[nsl], out0_h.at[nsl])

        @pl.when(cid == 1)
        def _():
            pltpu.sync_copy(acc.at[nsl], out1_h.at[nsl])

    return sc_scatter


CHUNKS = ((0, 96000), (96000, 64000))


def _forward_impl(vdata, edata, connectivity, params):
    f32 = jnp.float32
    src = connectivity[0]
    dst = connectivity[1]

    pce, pcn = params["core_edge"], params["core_node"]
    W1ce, W1cn = pce["W1"], pcn["W1"]
    A_e0, A_e = W1ce[0:LAT], W1ce[LAT:2 * LAT]
    A_v0s, A_vs = W1ce[2 * LAT:3 * LAT], W1ce[3 * LAT:4 * LAT]
    A_v0d, A_vd = W1ce[4 * LAT:5 * LAT], W1ce[5 * LAT:6 * LAT]
    B_v0, B_v, B_agg = W1cn[0:LAT], W1cn[LAT:2 * LAT], W1cn[2 * LAT:3 * LAT]

    def r2(x):
        return x.reshape(1, -1)

    en_, ee_ = params["enc_node"], params["enc_edge"]
    dn_, de_ = params["dec_node"], params["dec_edge"]

    w16 = _w_spec((LAT, HID))
    w1h = _w_spec((1, HID))
    w1l = _w_spec((1, LAT))

    ns, nd, pn, f_s, f_d, f_n = _tc_call(
        _enc_node_body, N_NODES // BN,
        in_specs=[_row_spec(BN, LAT), _w_spec((LAT, HID)), w1h,
                  _w_spec((HID, LAT)), w1l, w1l, w1l,
                  w16, w16, w16, w16, w16, w16],
        out_shapes=[jax.ShapeDtypeStruct((N_NODES, HID), f32)] * 6,
        out_specs=[_row_spec(BN, HID)] * 6,
        name="tc_enc_node",
    )(vdata, en_["W1"], r2(en_["b1"]), en_["W2"], r2(en_["b2"]),
      r2(en_["g"]), r2(en_["bt"]),
      A_vs, A_vd, B_v, A_v0s, A_v0d, B_v0)

    eye8 = jnp.eye(8, dtype=f32)
    k8 = lambda w: jnp.kron(eye8, w)
    t8 = lambda b: jnp.tile(b, 8).reshape(1, -1)
    g3ce = pce["g"].reshape(1, 1, LAT)
    bt3ce = pce["bt"].reshape(1, 1, LAT)
    edp = jnp.reshape(edata, (EP8, 8 * HID))
    wp = _w_spec((BP, 8 * HID))
    wpl = _w_spec((BP, LAT))

    pe0, pec = _tc_call(
        _enc_edge_body, N_EDGES // BE,
        in_specs=[_row_spec(BP, 8 * HID), _w_spec((8 * HID, 8 * HID)),
                  _w_spec((1, 8 * HID)), _w_spec((8 * HID, 8 * LAT)),
                  _w_spec((1, 8 * LAT)), _w_spec((1, 1, LAT)),
                  _w_spec((1, 1, LAT)), _w_spec((8 * LAT, 8 * HID)),
                  _w_spec((8 * LAT, 8 * HID))],
        out_shapes=[jax.ShapeDtypeStruct((EP8, 8 * HID), f32)] * 2,
        out_specs=[_row_spec(BP, 8 * HID)] * 2,
        name="tc_enc_edge",
    )(edp, k8(ee_["W1"]), t8(ee_["b1"]), k8(ee_["W2"]), t8(ee_["b2"]),
      ee_["g"].reshape(1, 1, LAT), ee_["bt"].reshape(1, 1, LAT),
      k8(A_e0), k8(A_e))

    zeros_nh = jnp.zeros((N_NODES, HID), f32)
    zeros_nl = jnp.zeros((N_NODES, LAT), f32)

    def do_gather(eb, ec, ns_, nd_):
        out = _sc_gather_k(eb, ec)(ns_, nd_, src, dst)
        return out[0] if isinstance(out, (list, tuple)) else out

    def do_count():
        return _sc_count_k()(dst, zeros_nh)

    def do_scatter(eb, ec, pb, x):
        return _sc_scatter_k(eb, ec, pb)(x, dst, zeros_nl)

    cntp = do_count()

    def edge_specs(e_base, pec_is_half):
        off = e_base // BE
        full = pl.BlockSpec((BP, 8 * HID), lambda i, o=off: (i + o, 0))
        half = _row_spec(BP, 8 * HID)
        return [full, half if pec_is_half else full, half]

    ew = [t8(pce["b1"]), k8(pce["W2"]), t8(pce["b2"]), g3ce, bt3ce]
    ew_specs = [_w_spec((1, 8 * HID)), _w_spec((8 * HID, 8 * LAT)),
                _w_spec((1, 8 * LAT)), _w_spec((1, 1, LAT)),
                _w_spec((1, 1, LAT))]

    Aeb = k8(A_e)
    pec_halves = None
    for t in range(CORE_STEPS - 1):
        hsums = [jnp.reshape(do_gather(eb, ec, ns, nd), (ec // 8, 8 * HID))
                 for (eb, ec) in CHUNKS]
        ens, aggs, pec_new = [], [], []
        for ci, (eb, ec) in enumerate(CHUNKS):
            pec_in = pec if pec_halves is None else pec_halves[ci]
            en, pc = _tc_call(
                _edge_step_body, ec // BE,
                in_specs=edge_specs(eb, pec_halves is not None) + ew_specs
                + [_w_spec((8 * LAT, 8 * HID))],
                out_shapes=[jax.ShapeDtypeStruct((ec, LAT), f32),
                            jax.ShapeDtypeStruct((ec // 8, 8 * HID), f32)],
                out_specs=[_row_spec(BE, LAT), _row_spec(BP, 8 * HID)],
                name=f"tc_edge_step{t}_{ci}",
            )(pe0, pec_in, hsums[ci], *ew, Aeb)
            ens.append(en)
            pec_new.append(pc)
            aggs.append(do_scatter(eb, ec, 0, en))
        pec_halves = pec_new

        ns, nd, pn = _tc_call(
            _node_step_body, N_NODES // BN,
            in_specs=[_row_spec(BN, HID), _row_spec(BN, LAT),
                      _row_spec(BN, LAT), _row_spec(BN, LAT),
                      _row_spec(BN, LAT), _row_spec(BN, HID),
                      _row_spec(BN, HID), _row_spec(BN, HID),
                      _row_spec(BN, HID), _row_spec(BN, HID),
                      w1h, _w_spec((HID, LAT)), w1l, w1l, w1l,
                      w16, w16, w16, w16],
            out_shapes=[jax.ShapeDtypeStruct((N_NODES, HID), f32)] * 3,
            out_specs=[_row_spec(BN, HID)] * 3,
            name=f"tc_node_step{t}",
        )(pn, aggs[0][0], aggs[0][1], aggs[1][0], aggs[1][1],
          cntp[0], cntp[1], f_s, f_d, f_n,
          r2(pcn["b1"]), pcn["W2"], r2(pcn["b2"]), r2(pcn["g"]),
          r2(pcn["bt"]), B_agg, A_vs, A_vd, B_v)

    hs3 = [jnp.reshape(do_gather(eb, ec, ns, nd), (ec // 8, 8 * HID))
           for (eb, ec) in CHUNKS]
    pec3 = jnp.concatenate(pec_halves, axis=0)
    nb0 = CHUNKS[0][1] // BE
    en3 = _tc_call(
        _edge_core_body, N_EDGES // BE,
        in_specs=[_row_spec(BP, 8 * HID), _row_spec(BP, 8 * HID),
                  pl.BlockSpec((BP, 8 * HID),
                               lambda i: (jnp.minimum(i, nb0 - 1), 0)),
                  pl.BlockSpec((BP, 8 * HID),
                               lambda i: (jnp.maximum(i - nb0, 0), 0))]
        + ew_specs,
        out_shapes=jax.ShapeDtypeStruct((N_EDGES, LAT), f32),
        out_specs=_row_spec(BE, LAT),
        name="tc_edge_core_last",
    )(pe0, pec3, hs3[0], hs3[1], *ew)

    aggs3 = [do_scatter(eb, ec, eb, en3) for (eb, ec) in CHUNKS]

    e_out = _tc_call(
        _edge_dec_body, N_EDGES // BE,
        in_specs=[_row_spec(BE, LAT),
                  _w_spec((LAT, HID)), w1h, _w_spec((HID, LAT)),
                  w1l, w1l, w1l, _w_spec((LAT, LAT)), w1l],
        out_shapes=jax.ShapeDtypeStruct((N_EDGES, LAT), f32),
        out_specs=_row_spec(BE, LAT),
        name="tc_edge_dec",
    )(en3, de_["W1"], r2(de_["b1"]), de_["W2"], r2(de_["b2"]),
      r2(de_["g"]), r2(de_["bt"]),
      params["dec_edge_out_W"], r2(params["dec_edge_out_b"]))

    v_out = _tc_call(
        _node_last_body, N_NODES // BN,
        in_specs=[_row_spec(BN, HID), _row_spec(BN, LAT),
                  _row_spec(BN, LAT), _row_spec(BN, LAT),
                  _row_spec(BN, LAT), _row_spec(BN, HID),
                  _row_spec(BN, HID),
                  w1h, _w_spec((HID, LAT)), w1l, w1l, w1l, w16,
                  _w_spec((LAT, HID)), w1h, _w_spec((HID, LAT)),
                  w1l, w1l, w1l, _w_spec((LAT, LAT)), w1l],
        out_shapes=jax.ShapeDtypeStruct((N_NODES, LAT), f32),
        out_specs=_row_spec(BN, LAT),
        name="tc_node_last",
    )(pn, aggs3[0][0], aggs3[0][1], aggs3[1][0], aggs3[1][1],
      cntp[0], cntp[1],
      r2(pcn["b1"]), pcn["W2"], r2(pcn["b2"]), r2(pcn["g"]),
      r2(pcn["bt"]), B_agg,
      dn_["W1"], r2(dn_["b1"]), dn_["W2"], r2(dn_["b2"]),
      r2(dn_["g"]), r2(dn_["bt"]),
      params["dec_node_out_W"], r2(params["dec_node_out_b"]))

    return (v_out, e_out)


def kernel(vdata, edata, connectivity, cdata, metadata, params):
    return _forward_impl(vdata, edata, connectivity, params)
